# Initial kernel scaffold; baseline (speedup 1.0000x reference)
import jax
import jax.numpy as jnp
from jax.experimental import pallas as pl


def kernel(d_sim, m_sim, W_d1, W_m1, W_d2, W_m2, s, Wd_fc, bd_fc, Wm_fc, bm_fc, Wp, bp, edge_index, diseases, mirnas):
    raise NotImplementedError("write your pallas kernel here")


# speedup vs baseline: 2.0834x; 2.0834x over previous
import functools

import jax
import jax.numpy as jnp
from jax import lax
from jax.experimental import pallas as pl
from jax.experimental.pallas import tpu as pltpu
from jax.experimental.pallas import tpu_sc as plsc

N = 10000
ND = 4000
E = 160000
D = 256
F = 256
OUT = 128
KHOP = 3
BP = 16384
SLOPE = 0.2

NC = 2
NS = 16
L = 16
NW = NC * NS

EPAD = 163840
EWA = EPAD // NW
EWS = EPAD // NS
CH = 64
NB = 10240
NSLC = NB // NS
HF = 128

_mesh = plsc.VectorSubcoreMesh(
    core_axis_name="c", subcore_axis_name="s", num_cores=NC, num_subcores=NS)


def _elu(x):
  return jnp.where(x > 0, x, jnp.exp(jnp.minimum(x, 0.0)) - 1.0)


def _sigmoid(x):
  return 1.0 / (1.0 + jnp.exp(-x))


def _f32(*shape):
  return jax.ShapeDtypeStruct(shape, jnp.float32)


def _wid():
  c = lax.axis_index("c")
  s = lax.axis_index("s")
  return c, s, c * NS + s


def _iota16():
  return lax.broadcasted_iota(jnp.int32, (L,), 0)


def _vtake(x, idx):
  return x.at[idx].get(mode="promise_in_bounds")


@functools.partial(
    pl.kernel,
    out_type=(_f32(EPAD), _f32(NW, NB), _f32(NC, NB)),
    mesh=_mesh,
    compiler_params=pltpu.CompilerParams(needs_layout_passes=False),
    scratch_types=[
        pltpu.VMEM((CH,), jnp.int32),
        pltpu.VMEM((CH,), jnp.int32),
        pltpu.VMEM((CH,), jnp.int32),
        pltpu.VMEM((CH,), jnp.int32),
        pltpu.VMEM((CH,), jnp.int32),
        pltpu.VMEM((CH, HF), jnp.float32),
        pltpu.VMEM((CH, HF), jnp.float32),
        pltpu.VMEM((CH, HF), jnp.float32),
        pltpu.VMEM((CH, HF), jnp.float32),
        pltpu.VMEM((CH,), jnp.float32),
        pltpu.VMEM((CH,), jnp.float32),
        pltpu.VMEM((NB,), jnp.float32),
        pltpu.VMEM_SHARED((NB,), jnp.float32),
        pltpu.SemaphoreType.DMA,
    ],
)
def _edge_scores(zh, idx4, dst_pad, zeros_nb, ones_ch, e_out, maxpart,
                 degpart, ia, ib, ic, idd, dstb, rsl, rsh, rdl, rdh, ebuf,
                 onesb, mbins, degsh, sem):
  cid, sid, wid = _wid()
  pltpu.sync_copy(zeros_nb, mbins)
  pltpu.sync_copy(ones_ch, onesb)
  pltpu.sync_copy(mbins.at[pl.ds(0, NSLC)], degsh.at[pl.ds(sid * NSLC, NSLC)])
  plsc.subcore_barrier()

  def chunk(i, carry):
    base = wid * EWA + i * CH
    pltpu.sync_copy(idx4.at[0, pl.ds(base, CH)], ia)
    pltpu.sync_copy(idx4.at[1, pl.ds(base, CH)], ib)
    pltpu.sync_copy(idx4.at[2, pl.ds(base, CH)], ic)
    pltpu.sync_copy(idx4.at[3, pl.ds(base, CH)], idd)
    pltpu.sync_copy(dst_pad.at[pl.ds(base, CH)], dstb)
    c1 = pltpu.async_copy(zh.at[ia], rsl, sem)
    c2 = pltpu.async_copy(zh.at[ib], rsh, sem)
    c3 = pltpu.async_copy(zh.at[ic], rdl, sem)
    c4 = pltpu.async_copy(zh.at[idd], rdh, sem)
    c1.wait()
    c2.wait()
    c3.wait()
    c4.wait()

    iota = _iota16()

    def grp(g, carry2):

      def edge(j, ev):
        row = g * L + j
        acc = jnp.zeros((L,), jnp.float32)
        for m in range(HF // L):
          sl2 = pl.ds(m * L, L)
          acc = acc + rsl[row, sl2] * rdl[row, sl2]
          acc = acc + rsh[row, sl2] * rdh[row, sl2]
        dot = jnp.sum(acc)
        e = jnp.where(dot > 0, dot, SLOPE * dot)
        return jnp.where(iota == j, e, ev)

      ev = lax.fori_loop(0, L, edge, jnp.zeros((L,), jnp.float32))
      ebuf[pl.ds(g * L, L)] = ev

      ks, vs = plsc.sort_key_val(dstb[pl.ds(g * L, L)], ev)
      for sh in (1, 2, 4, 8):
        pidx = jnp.maximum(iota - sh, 0)
        kp = _vtake(ks, pidx)
        vp = _vtake(vs, pidx)
        vs = jnp.where((kp == ks) & (iota >= sh), jnp.maximum(vs, vp), vs)
      knext = _vtake(ks, jnp.minimum(iota + 1, L - 1))
      is_last = (ks != knext) | (iota == L - 1)
      cur = plsc.load_gather(mbins, [ks])
      plsc.store_scatter(mbins, [ks], jnp.maximum(cur, vs), mask=is_last)
      return carry2

    lax.fori_loop(0, CH // L, grp, 0)

    pltpu.sync_copy(onesb, degsh.at[dstb], add=True)
    pltpu.sync_copy(ebuf, e_out.at[pl.ds(base, CH)])
    return carry

  lax.fori_loop(0, EWA // CH, chunk, 0)
  pltpu.sync_copy(mbins, maxpart.at[wid])
  plsc.subcore_barrier()

  @pl.when(sid == 0)
  def _():
    pltpu.sync_copy(degsh, degpart.at[cid])


@functools.partial(
    pl.kernel,
    out_type=(_f32(NC, NB, HF), _f32(NC, NB)),
    mesh=_mesh,
    compiler_params=pltpu.CompilerParams(needs_layout_passes=False),
    scratch_types=[
        pltpu.VMEM((CH,), jnp.int32),
        pltpu.VMEM((CH,), jnp.int32),
        pltpu.VMEM((CH, HF), jnp.float32),
        pltpu.VMEM((CH, HF), jnp.float32),
        pltpu.VMEM((CH,), jnp.float32),
        pltpu.VMEM((CH,), jnp.float32),
        pltpu.VMEM((NB,), jnp.float32),
        pltpu.VMEM_SHARED((NB, HF), jnp.float32),
        pltpu.VMEM_SHARED((NB,), jnp.float32),
        pltpu.SemaphoreType.DMA,
    ],
)
def _attn_aggregate(zh, srcz, dst_pad, e_in, cvec, zeros_nb, zeros64,
                    num_out, den_out, idxb, dstb, rows, scal, ebuf, exbuf,
                    cbins, numsh, densh, sem):
  cid, sid, _ = _wid()
  pltpu.sync_copy(cvec, cbins)
  pltpu.sync_copy(zeros_nb.at[pl.ds(sid * NSLC, NSLC)],
                  densh.at[pl.ds(sid * NSLC, NSLC)])

  def zinit(k, carry):
    pltpu.sync_copy(zeros64, numsh.at[pl.ds(sid * NSLC + k * CH, CH)])
    return carry

  lax.fori_loop(0, NSLC // CH, zinit, 0)
  plsc.subcore_barrier()

  def chunk(i, carry):
    base = sid * EWS + i * CH
    pltpu.sync_copy(srcz.at[cid, pl.ds(base, CH)], idxb)
    pltpu.sync_copy(dst_pad.at[pl.ds(base, CH)], dstb)
    pltpu.sync_copy(e_in.at[pl.ds(base, CH)], ebuf)
    pltpu.async_copy(zh.at[idxb], rows, sem).wait()
    def grp(g, carry2):
      sl = pl.ds(g * L, L)
      cg = plsc.load_gather(cbins, [dstb[sl]])
      exv = jnp.exp(ebuf[sl] - cg)
      exbuf[sl] = exv

      def edge(j, carry3):
        row = g * L + j
        sv = _vtake(exv, jnp.full((L,), j, jnp.int32))
        for m in range(HF // L):
          sl2 = pl.ds(m * L, L)
          scal[row, sl2] = rows[row, sl2] * sv
        return carry3

      return lax.fori_loop(0, L, edge, carry2)

    lax.fori_loop(0, CH // L, grp, 0)
    pltpu.sync_copy(scal, numsh.at[dstb], add=True)
    pltpu.sync_copy(exbuf, densh.at[dstb], add=True)
    return carry

  lax.fori_loop(0, EWS // CH, chunk, 0)
  plsc.subcore_barrier()
  pltpu.sync_copy(numsh.at[pl.ds(sid * NSLC, NSLC)],
                  num_out.at[cid, pl.ds(sid * NSLC, NSLC)])

  @pl.when(sid == 0)
  def _():
    pltpu.sync_copy(densh, den_out.at[cid])


@functools.partial(
    pl.kernel,
    out_type=(_f32(KHOP, NC * NB, HF), _f32(NC * NB, HF), _f32(NC * NB, HF)),
    mesh=_mesh,
    compiler_params=pltpu.CompilerParams(needs_layout_passes=False),
    scratch_types=[
        pltpu.VMEM((CH,), jnp.int32),
        pltpu.VMEM((CH,), jnp.int32),
        pltpu.VMEM((CH, HF), jnp.float32),
        pltpu.VMEM((CH, HF), jnp.float32),
        pltpu.VMEM((CH, HF), jnp.float32),
        pltpu.VMEM((NSLC,), jnp.float32),
        pltpu.VMEM_SHARED((NB, HF), jnp.float32),
        pltpu.SemaphoreType.DMA,
    ],
)
def _dagnn(g0, srcg, dst_pad, normv, zeros64, f_all, gta, gtb, idxb, dstb,
           rows, fbuf, gbuf, normb, binsh, sem):
  cid, sid, _ = _wid()
  pltpu.sync_copy(normv.at[pl.ds(sid * NSLC, NSLC)], normb)
  tabs = (g0, gta, gtb)
  for r in range(KHOP):
    gtab = tabs[r]

    def zinit(k, carry):
      pltpu.sync_copy(zeros64, binsh.at[pl.ds(sid * NSLC + k * CH, CH)])
      return carry

    lax.fori_loop(0, NSLC // CH, zinit, 0)
    plsc.subcore_barrier()

    def chunk(i, carry):
      base = sid * EWS + i * CH
      pltpu.sync_copy(srcg.at[cid, pl.ds(base, CH)], idxb)
      pltpu.sync_copy(dst_pad.at[pl.ds(base, CH)], dstb)
      pltpu.async_copy(gtab.at[idxb], rows, sem).wait()
      pltpu.sync_copy(rows, binsh.at[dstb], add=True)
      return carry

    lax.fori_loop(0, EWS // CH, chunk, 0)
    plsc.subcore_barrier()

    def slice_k(k, carry):
      off = sid * NSLC + k * CH
      pltpu.sync_copy(binsh.at[pl.ds(off, CH)], rows)

      def grp(g, carry2):
        nv = normb[pl.ds(k * CH + g * L, L)]

        def node(j, carry3):
          row = g * L + j
          sv = _vtake(nv, jnp.full((L,), j, jnp.int32))
          for m in range(HF // L):
            sl = pl.ds(m * L, L)
            v = rows[row, sl] * sv
            fbuf[row, sl] = v
            gbuf[row, sl] = v * sv
          return carry3

        return lax.fori_loop(0, L, node, carry2)

      lax.fori_loop(0, CH // L, grp, 0)
      pltpu.sync_copy(fbuf, f_all.at[r, pl.ds(cid * NB + off, CH)])
      if r == 0:
        pltpu.sync_copy(gbuf, gta.at[pl.ds(cid * NB + off, CH)])
      elif r == 1:
        pltpu.sync_copy(gbuf, gtb.at[pl.ds(cid * NB + off, CH)])
      return carry

    lax.fori_loop(0, NSLC // CH, slice_k, 0)
    plsc.subcore_barrier()


@functools.partial(
    pl.kernel,
    out_type=_f32(BP),
    mesh=_mesh,
    compiler_params=pltpu.CompilerParams(needs_layout_passes=False),
    scratch_types=[
        pltpu.VMEM((CH,), jnp.int32),
        pltpu.VMEM((CH,), jnp.int32),
        pltpu.VMEM((CH, OUT), jnp.float32),
        pltpu.VMEM((CH, OUT), jnp.float32),
        pltpu.VMEM((OUT,), jnp.float32),
        pltpu.VMEM((OUT,), jnp.float32),
        pltpu.VMEM((L,), jnp.float32),
        pltpu.VMEM((CH,), jnp.float32),
        pltpu.SemaphoreType.DMA,
    ],
)
def _pair_scores(h, dis, mir, wpd, wpm, bp16, o_out, dib, mib, hrd, hrm,
                 wdb, wmb, bpb, obuf, sem):
  _, _, wid = _wid()
  pltpu.sync_copy(wpd, wdb)
  pltpu.sync_copy(wpm, wmb)
  pltpu.sync_copy(bp16, bpb)
  npairs = BP // NW

  def chunk(i, carry):
    base = wid * npairs + i * CH
    pltpu.sync_copy(dis.at[pl.ds(base, CH)], dib)
    pltpu.sync_copy(mir.at[pl.ds(base, CH)], mib)
    c1 = pltpu.async_copy(h.at[dib], hrd, sem)
    c2 = pltpu.async_copy(h.at[mib], hrm, sem)
    c1.wait()
    c2.wait()

    iota = _iota16()
    for g in range(CH // L):

      def pair(j, tv, _g=g):
        row = _g * L + j
        acc = jnp.zeros((L,), jnp.float32)
        for m in range(OUT // L):
          sl2 = pl.ds(m * L, L)
          acc = acc + hrd[row, sl2] * wdb[sl2]
          acc = acc + hrm[row, sl2] * wmb[sl2]
        return jnp.where(iota == j, jnp.sum(acc), tv)

      tv = lax.fori_loop(0, L, pair, jnp.zeros((L,), jnp.float32))
      t = tv + bpb[...]
      obuf[pl.ds(g * L, L)] = 1.0 / (1.0 + jnp.exp(-t))
    pltpu.sync_copy(obuf, o_out.at[pl.ds(base, CH)])
    return carry

  lax.fori_loop(0, npairs // CH, chunk, 0)


def _k1_body(d_ref, m_ref, wd_ref, wm_ref, zh_ref):
  p = pl.program_id(0)
  is_d = (p % 10) < (ND // 1000)
  x = jnp.where(is_d, d_ref[...], m_ref[...])
  w = jnp.where(is_d, wd_ref[...], wm_ref[...])
  zh_ref[...] = jnp.dot(x, w, preferred_element_type=jnp.float32)


def _node_transform(d_sim, m_sim, wd, wm):
  return pl.pallas_call(
      _k1_body,
      grid=(20,),
      in_specs=[
          pl.BlockSpec((1000, D), lambda g: (g % 10, 0)),
          pl.BlockSpec((1000, D), lambda g: (g % 10, 0)),
          pl.BlockSpec((D, HF), lambda g: (0, g // 10)),
          pl.BlockSpec((D, HF), lambda g: (0, g // 10)),
      ],
      out_specs=pl.BlockSpec((1000, HF), lambda g: (g, 0)),
      out_shape=_f32(2 * N, HF),
  )(d_sim, m_sim, wd, wm)


def _k2_body(mx_ref, dg_ref, c_ref, n_ref):
  c = jnp.max(mx_ref[...], axis=0)
  c_ref[...] = jnp.maximum(c, 0.0)
  deg = jnp.sum(dg_ref[...], axis=0)
  n_ref[...] = jnp.where(deg > 0, lax.rsqrt(jnp.maximum(deg, 1e-30)), 0.0)


def _combine_stats(maxpart, degpart):
  return pl.pallas_call(
      _k2_body,
      out_shape=(_f32(NB), _f32(NB)),
  )(maxpart, degpart)


def _k3_body(num_ref, den_ref, nrm_ref, ft_ref, g0_ref):
  i = pl.program_id(0)
  den = den_ref[0, pl.ds(i % 8 * 1280, 1280)]
  nrm = nrm_ref[pl.ds(i % 8 * 1280, 1280)]
  den = jnp.where(den > 0, den, 1.0)
  feats = _elu(num_ref[...] / den[:, None])
  ft_ref[...] = feats
  g0_ref[...] = feats * nrm[:, None]


def _feats_g0(num, den, normv):
  return pl.pallas_call(
      _k3_body,
      grid=(16,),
      in_specs=[
          pl.BlockSpec((1280, HF), lambda i: (i, 0)),
          pl.BlockSpec((NC, NB), lambda i: (0, 0)),
          pl.BlockSpec((NB,), lambda i: (0,)),
      ],
      out_specs=[
          pl.BlockSpec((1280, HF), lambda i: (i, 0)),
          pl.BlockSpec((1280, HF), lambda i: (i, 0)),
      ],
      out_shape=(_f32(NC * NB, HF), _f32(NC * NB, HF)),
  )(num.reshape(NC * NB, HF), den, normv)


def _k4_body(ft_ref, f1_ref, f2_ref, f3_ref, d_ref, m_ref, s_ref, wd_ref,
             bd_ref, wm_ref, bm_ref, h_ref):
  p = pl.program_id(0)
  hout = jnp.zeros((1000, F), jnp.float32)
  for ref in (ft_ref, f1_ref, f2_ref, f3_ref):
    hk = jnp.concatenate([ref[0], ref[1]], axis=1)
    sk = _sigmoid(jnp.dot(hk, s_ref[...],
                                preferred_element_type=jnp.float32))
    hout = hout + sk[:, None] * hk
  is_d = p < (ND // 1000)
  sim = jnp.where(is_d, d_ref[...], m_ref[...])
  w = jnp.where(is_d, wd_ref[...], wm_ref[...])
  b = jnp.where(is_d, bd_ref[...], bm_ref[...])
  x = jnp.concatenate([hout, sim], axis=1)
  h_ref[...] = _elu(
      jnp.dot(x, w, preferred_element_type=jnp.float32) + b[None, :])


def _final_mlp(ft, f1, f2, f3, d_sim, m_sim, s, wd_fc, bd_fc, wm_fc, bm_fc):
  fspec = pl.BlockSpec((NC, 1000, HF), lambda i: (0, i, 0))
  return pl.pallas_call(
      _k4_body,
      grid=(10,),
      in_specs=[
          fspec, fspec, fspec, fspec,
          pl.BlockSpec((1000, D), lambda i: (i, 0)),
          pl.BlockSpec((1000, D), lambda i: (i, 0)),
          pl.BlockSpec((F,), lambda i: (0,)),
          pl.BlockSpec((F + D, OUT), lambda i: (0, 0)),
          pl.BlockSpec((OUT,), lambda i: (0,)),
          pl.BlockSpec((F + D, OUT), lambda i: (0, 0)),
          pl.BlockSpec((OUT,), lambda i: (0,)),
      ],
      out_specs=pl.BlockSpec((1000, OUT), lambda i: (i, 0)),
      out_shape=_f32(N, OUT),
  )(ft, f1, f2, f3, d_sim, m_sim, s, wd_fc, bd_fc, wm_fc, bm_fc)


def kernel(d_sim, m_sim, W_d1, W_m1, W_d2, W_m2, s, Wd_fc, bd_fc, Wm_fc,
           bm_fc, Wp, bp, edge_index, diseases, mirnas):
  src = edge_index[0]
  dst = edge_index[1]
  pad = EPAD - E
  src_pad = jnp.concatenate([src, jnp.zeros((pad,), jnp.int32)])
  dst_pad = jnp.concatenate([dst, jnp.full((pad,), N, jnp.int32)])
  dstg = jnp.minimum(dst_pad, N - 1)
  idx4 = jnp.stack([src_pad, src_pad + N, dstg, dstg + N])
  srcz = jnp.stack([src_pad, src_pad + N])
  srcg = jnp.stack([src_pad, src_pad + NB])
  zeros_nb = jnp.zeros((NB,), jnp.float32)
  zeros64 = jnp.zeros((CH, HF), jnp.float32)
  ones_ch = jnp.ones((CH,), jnp.float32)
  wpd = Wp[:OUT, 0]
  wpm = Wp[OUT:, 0]
  bp16 = jnp.full((L,), 0.0, jnp.float32) + bp[0]

  zh = _node_transform(d_sim, m_sim, W_d2, W_m2)
  e_pad, maxpart, degpart = _edge_scores(zh, idx4, dst_pad, zeros_nb, ones_ch)
  cvec, normv = _combine_stats(maxpart, degpart)
  num, den = _attn_aggregate(zh, srcz, dst_pad, e_pad, cvec, zeros_nb,
                             zeros64)
  feats, g0 = _feats_g0(num, den, normv)
  f_all, _, _ = _dagnn(g0, srcg, dst_pad, normv, zeros64)
  ftr = feats.reshape(NC, NB, HF)
  f1 = f_all[0].reshape(NC, NB, HF)
  f2 = f_all[1].reshape(NC, NB, HF)
  f3 = f_all[2].reshape(NC, NB, HF)
  h = _final_mlp(ftr, f1, f2, f3, d_sim, m_sim, s[:, 0], Wd_fc, bd_fc,
                 Wm_fc, bm_fc)
  o = _pair_scores(h, diseases, mirnas, wpd, wpm, bp16)
  return o.reshape(BP, 1)


# speedup vs baseline: 3.0528x; 1.4653x over previous
import functools

import jax
import jax.numpy as jnp
from jax import lax
from jax.experimental import pallas as pl
from jax.experimental.pallas import tpu as pltpu
from jax.experimental.pallas import tpu_sc as plsc

N = 10000
ND = 4000
E = 160000
D = 256
F = 256
OUT = 128
KHOP = 3
BP = 16384
SLOPE = 0.2

NC = 2
NS = 16
L = 16
NW = NC * NS

EPAD = 163840
EWA = EPAD // NW
EWS = EPAD // NS
CH = 64
CHA = 64
NCHA = EWA // CHA
CHB = 128
NCHB = EWS // CHB
CHC = 128
NCHC = EWS // CHC
NB = 10240
NSLC = NB // NS
HF = 128

_mesh = plsc.VectorSubcoreMesh(
    core_axis_name="c", subcore_axis_name="s", num_cores=NC, num_subcores=NS)


def _elu(x):
  return jnp.where(x > 0, x, jnp.exp(jnp.minimum(x, 0.0)) - 1.0)


def _sigmoid(x):
  return 1.0 / (1.0 + jnp.exp(-x))


def _f32(*shape):
  return jax.ShapeDtypeStruct(shape, jnp.float32)


def _wid():
  c = lax.axis_index("c")
  s = lax.axis_index("s")
  return c, s, c * NS + s


def _iota16():
  return lax.broadcasted_iota(jnp.int32, (L,), 0)


def _vtake(x, idx):
  return x.at[idx].get(mode="promise_in_bounds")


def _runs(ks, vs, combine):
  iota = _iota16()
  for sh in (1, 2, 4, 8):
    pidx = jnp.maximum(iota - sh, 0)
    kp = _vtake(ks, pidx)
    vp = _vtake(vs, pidx)
    vs = jnp.where((kp == ks) & (iota >= sh), combine(vs, vp), vs)
  knext = _vtake(ks, jnp.minimum(iota + 1, L - 1))
  is_last = (ks != knext) | (iota == L - 1)
  return vs, is_last


@functools.partial(
    pl.kernel,
    out_type=(_f32(EPAD), _f32(NW, NB), _f32(NW, NB)),
    mesh=_mesh,
    compiler_params=pltpu.CompilerParams(needs_layout_passes=False),
    scratch_types=[
        pltpu.VMEM((NCHA, CHA), jnp.int32),
        pltpu.VMEM((NCHA, CHA), jnp.int32),
        pltpu.VMEM((NCHA, CHA), jnp.int32),
        pltpu.VMEM((EWA,), jnp.float32),
        pltpu.VMEM((NB,), jnp.float32),
        pltpu.VMEM((NB,), jnp.float32),
        pltpu.VMEM((CHA, D), jnp.float32),
        pltpu.VMEM((CHA, D), jnp.float32),
        pltpu.VMEM((CHA, D), jnp.float32),
        pltpu.VMEM((CHA, D), jnp.float32),
        pltpu.SemaphoreType.DMA,
        pltpu.SemaphoreType.DMA,
    ],
)
def _edge_scores(z, srcA, dgA, dstA, zeros_nb, e_out, maxpart, degpart,
                 sA, gA, kA, ebuf, mbins, dbins, sr0, dr0, sr1, dr1, g0, g1):
  cid, sid, wid = _wid()
  pltpu.sync_copy(srcA.at[wid], sA)
  pltpu.sync_copy(dgA.at[wid], gA)
  pltpu.sync_copy(dstA.at[wid], kA)
  pltpu.sync_copy(zeros_nb, mbins)
  pltpu.sync_copy(zeros_nb, dbins)
  iota = _iota16()
  ones = jnp.ones((L,), jnp.float32)

  def process(k, sr, dr):
    def grp(g, carry2):
      def edge(j, ev):
        row = g * L + j
        acc = jnp.zeros((L,), jnp.float32)
        for m in range(D // L):
          sl2 = pl.ds(m * L, L)
          acc = acc + sr[row, sl2] * dr[row, sl2]
        dot = jnp.sum(acc)
        e = jnp.where(dot > 0, dot, SLOPE * dot)
        return jnp.where(iota == j, e, ev)

      ev = lax.fori_loop(0, L, edge, jnp.zeros((L,), jnp.float32))
      ebuf[pl.ds(k * CHA + g * L, L)] = ev
      ks, vs = plsc.sort_key_val(kA[k, pl.ds(g * L, L)], ev)
      vmax, is_last = _runs(ks, vs, jnp.maximum)
      cur = plsc.load_gather(mbins, [ks])
      plsc.store_scatter(mbins, [ks], jnp.maximum(cur, vmax), mask=is_last)
      cnt, _ = _runs(ks, ones, lambda a, b: a + b)
      dcur = plsc.load_gather(dbins, [ks])
      plsc.store_scatter(dbins, [ks], dcur + cnt, mask=is_last)
      return carry2

    lax.fori_loop(0, CHA // L, grp, 0)

  def pair(kk, carry):
    k0 = 2 * kk
    k1 = k0 + 1
    ds0 = pltpu.async_copy(z.at[sA.at[k0]], sr0, g0)
    dd0 = pltpu.async_copy(z.at[gA.at[k0]], dr0, g0)
    ds1 = pltpu.async_copy(z.at[sA.at[k1]], sr1, g1)
    dd1 = pltpu.async_copy(z.at[gA.at[k1]], dr1, g1)
    ds0.wait()
    dd0.wait()
    process(k0, sr0, dr0)
    ds1.wait()
    dd1.wait()
    process(k1, sr1, dr1)
    return carry

  lax.fori_loop(0, NCHA // 2, pair, 0)
  pltpu.sync_copy(ebuf, e_out.at[pl.ds(wid * EWA, EWA)])
  pltpu.sync_copy(mbins, maxpart.at[wid])
  pltpu.sync_copy(dbins, degpart.at[wid])


@functools.partial(
    pl.kernel,
    out_type=(_f32(EPAD), _f32(NW, NB)),
    mesh=_mesh,
    compiler_params=pltpu.CompilerParams(needs_layout_passes=False),
    scratch_types=[
        pltpu.VMEM((NCHA, CHA), jnp.int32),
        pltpu.VMEM((EWA,), jnp.float32),
        pltpu.VMEM((EWA,), jnp.float32),
        pltpu.VMEM((NB,), jnp.float32),
        pltpu.VMEM((NB,), jnp.float32),
    ],
)
def _edge_weights(e_in, dstA, cvec, zeros_nb, ex_out, denpart, kA, e1, exb,
                  cbins, dbins):
  cid, sid, wid = _wid()
  pltpu.sync_copy(dstA.at[wid], kA)
  pltpu.sync_copy(e_in.at[pl.ds(wid * EWA, EWA)], e1)
  pltpu.sync_copy(cvec, cbins)
  pltpu.sync_copy(zeros_nb, dbins)

  def chunk(k, carry):
    def grp(g, carry2):
      dk = kA[k, pl.ds(g * L, L)]
      evv = e1[pl.ds(k * CHA + g * L, L)]
      cg = plsc.load_gather(cbins, [dk])
      exv = jnp.exp(evv - cg)
      exb[pl.ds(k * CHA + g * L, L)] = exv
      ks, xs = plsc.sort_key_val(dk, exv)
      ssum, is_last = _runs(ks, xs, lambda a, b: a + b)
      cur = plsc.load_gather(dbins, [ks])
      plsc.store_scatter(dbins, [ks], cur + ssum, mask=is_last)
      return carry2

    return lax.fori_loop(0, CHA // L, grp, carry)

  lax.fori_loop(0, NCHA, chunk, 0)
  pltpu.sync_copy(exb, ex_out.at[pl.ds(wid * EWA, EWA)])
  pltpu.sync_copy(dbins, denpart.at[wid])


@functools.partial(
    pl.kernel,
    out_type=_f32(NC, NB, HF),
    mesh=_mesh,
    compiler_params=pltpu.CompilerParams(needs_layout_passes=False),
    scratch_types=[
        pltpu.VMEM((CHB,), jnp.int32),
        pltpu.VMEM((CHB,), jnp.int32),
        pltpu.VMEM((CHB,), jnp.int32),
        pltpu.VMEM((CHB,), jnp.int32),
        pltpu.VMEM((CHB,), jnp.float32),
        pltpu.VMEM((CHB,), jnp.float32),
        pltpu.VMEM((CHB, HF), jnp.float32),
        pltpu.VMEM((CHB, HF), jnp.float32),
        pltpu.VMEM_SHARED((NB, HF), jnp.float32),
        pltpu.SemaphoreType.DMA,
        pltpu.SemaphoreType.DMA,
        pltpu.SemaphoreType.DMA,
        pltpu.SemaphoreType.DMA,
    ],
)
def _attn_aggregate(zh, srczB, dstB, exB, zeros_rows, num_out, si0, si1,
                    di0, di1, ex0, ex1, r0, r1, numsh, g0, g1, s0, s1):
  cid, sid, wid = _wid()
  pltpu.sync_copy(zeros_rows, numsh.at[pl.ds(sid * NSLC, NSLC)])
  plsc.subcore_barrier()

  def ldidx(k, si, di, exb):
    pltpu.sync_copy(srczB.at[cid, sid, k], si)
    pltpu.sync_copy(dstB.at[sid, k], di)
    pltpu.sync_copy(exB.at[sid, k], exb)

  def scale(r, exb):
    def grp(g, carry2):
      exv = exb[pl.ds(g * L, L)]

      def edge(j, carry3):
        row = g * L + j
        sv = _vtake(exv, jnp.full((L,), j, jnp.int32))
        for m in range(HF // L):
          sl2 = pl.ds(m * L, L)
          r[row, sl2] = r[row, sl2] * sv
        return carry3

      return lax.fori_loop(0, L, edge, carry2)

    lax.fori_loop(0, CHB // L, grp, 0)

  def pair(kk, carry):
    k0 = 2 * kk
    k1 = k0 + 1
    ldidx(k0, si0, di0, ex0)
    dg0 = pltpu.async_copy(zh.at[si0], r0, g0)
    ldidx(k1, si1, di1, ex1)
    dg1 = pltpu.async_copy(zh.at[si1], r1, g1)
    dg0.wait()
    scale(r0, ex0)
    d0 = pltpu.async_copy(r0, numsh.at[di0], s0, add=True)
    dg1.wait()
    scale(r1, ex1)
    d1 = pltpu.async_copy(r1, numsh.at[di1], s1, add=True)
    d0.wait()
    d1.wait()
    return carry

  lax.fori_loop(0, NCHB // 2, pair, 0)
  plsc.subcore_barrier()
  pltpu.sync_copy(numsh.at[pl.ds(sid * NSLC, NSLC)],
                  num_out.at[cid, pl.ds(sid * NSLC, NSLC)])


@functools.partial(
    pl.kernel,
    out_type=(_f32(KHOP, NC * NB, HF), _f32(NC * NB, HF), _f32(NC * NB, HF)),
    mesh=_mesh,
    compiler_params=pltpu.CompilerParams(needs_layout_passes=False),
    scratch_types=[
        pltpu.VMEM((CHC,), jnp.int32),
        pltpu.VMEM((CHC,), jnp.int32),
        pltpu.VMEM((CHC,), jnp.int32),
        pltpu.VMEM((CHC,), jnp.int32),
        pltpu.VMEM((NSLC,), jnp.float32),
        pltpu.VMEM((CHC, HF), jnp.float32),
        pltpu.VMEM((CHC, HF), jnp.float32),
        pltpu.VMEM_SHARED((NB, HF), jnp.float32),
        pltpu.SemaphoreType.DMA,
        pltpu.SemaphoreType.DMA,
        pltpu.SemaphoreType.DMA,
        pltpu.SemaphoreType.DMA,
    ],
)
def _dagnn(g0t, srcgC, dstC, normv, zeros_rows, f_all, gta, gtb, si0, si1,
           di0, di1, normb, r0, r1, binsh, g0, g1, s0, s1):
  cid, sid, _ = _wid()
  pltpu.sync_copy(normv.at[pl.ds(sid * NSLC, NSLC)], normb)
  tabs = (g0t, gta, gtb)
  for r in range(KHOP):
    gtab = tabs[r]
    pltpu.sync_copy(zeros_rows, binsh.at[pl.ds(sid * NSLC, NSLC)])
    plsc.subcore_barrier()

    def pair(kk, carry, gtab=gtab):
      k0 = 2 * kk
      k1 = k0 + 1
      pltpu.sync_copy(srcgC.at[cid, sid, k0], si0)
      pltpu.sync_copy(dstC.at[sid, k0], di0)
      dg0 = pltpu.async_copy(gtab.at[si0], r0, g0)
      pltpu.sync_copy(srcgC.at[cid, sid, k1], si1)
      pltpu.sync_copy(dstC.at[sid, k1], di1)
      dg1 = pltpu.async_copy(gtab.at[si1], r1, g1)
      dg0.wait()
      d0 = pltpu.async_copy(r0, binsh.at[di0], s0, add=True)
      dg1.wait()
      d1 = pltpu.async_copy(r1, binsh.at[di1], s1, add=True)
      d0.wait()
      d1.wait()
      return carry

    lax.fori_loop(0, NCHC // 2, pair, 0)
    plsc.subcore_barrier()

    def slice_k(kk, carry):
      off = sid * NSLC + kk * CH
      pltpu.sync_copy(binsh.at[pl.ds(off, CH)], r0.at[pl.ds(0, CH)])

      def grp(g, carry2):
        nv = normb[pl.ds(kk * CH + g * L, L)]

        def node(j, carry3):
          row = g * L + j
          sv = _vtake(nv, jnp.full((L,), j, jnp.int32))
          for m in range(HF // L):
            sl = pl.ds(m * L, L)
            v = r0[row, sl] * sv
            r1[row, sl] = v
            r1[CH + row, sl] = v * sv
          return carry3

        return lax.fori_loop(0, L, node, carry2)

      lax.fori_loop(0, CH // L, grp, 0)
      pltpu.sync_copy(r1.at[pl.ds(0, CH)],
                      f_all.at[r, pl.ds(cid * NB + off, CH)])
      if r == 0:
        pltpu.sync_copy(r1.at[pl.ds(CH, CH)],
                        gta.at[pl.ds(cid * NB + off, CH)])
      elif r == 1:
        pltpu.sync_copy(r1.at[pl.ds(CH, CH)],
                        gtb.at[pl.ds(cid * NB + off, CH)])
      return carry

    lax.fori_loop(0, NSLC // CH, slice_k, 0)
    plsc.subcore_barrier()


@functools.partial(
    pl.kernel,
    out_type=_f32(BP),
    mesh=_mesh,
    compiler_params=pltpu.CompilerParams(needs_layout_passes=False),
    scratch_types=[
        pltpu.VMEM((CH,), jnp.int32),
        pltpu.VMEM((CH,), jnp.int32),
        pltpu.VMEM((CH, OUT), jnp.float32),
        pltpu.VMEM((CH, OUT), jnp.float32),
        pltpu.VMEM((OUT,), jnp.float32),
        pltpu.VMEM((OUT,), jnp.float32),
        pltpu.VMEM((L,), jnp.float32),
        pltpu.VMEM((CH,), jnp.float32),
        pltpu.SemaphoreType.DMA,
    ],
)
def _pair_scores(h, dis, mir, wpd, wpm, bp16, o_out, dib, mib, hrd, hrm,
                 wdb, wmb, bpb, obuf, sem):
  _, _, wid = _wid()
  pltpu.sync_copy(wpd, wdb)
  pltpu.sync_copy(wpm, wmb)
  pltpu.sync_copy(bp16, bpb)
  npairs = BP // NW

  def chunk(i, carry):
    base = wid * npairs + i * CH
    pltpu.sync_copy(dis.at[pl.ds(base, CH)], dib)
    pltpu.sync_copy(mir.at[pl.ds(base, CH)], mib)
    c1 = pltpu.async_copy(h.at[dib], hrd, sem)
    c2 = pltpu.async_copy(h.at[mib], hrm, sem)
    c1.wait()
    c2.wait()

    iota = _iota16()
    for g in range(CH // L):

      def pair(j, tv, _g=g):
        row = _g * L + j
        acc = jnp.zeros((L,), jnp.float32)
        for m in range(OUT // L):
          sl2 = pl.ds(m * L, L)
          acc = acc + hrd[row, sl2] * wdb[sl2]
          acc = acc + hrm[row, sl2] * wmb[sl2]
        return jnp.where(iota == j, jnp.sum(acc), tv)

      tv = lax.fori_loop(0, L, pair, jnp.zeros((L,), jnp.float32))
      t = tv + bpb[...]
      obuf[pl.ds(g * L, L)] = 1.0 / (1.0 + jnp.exp(-t))
    pltpu.sync_copy(obuf, o_out.at[pl.ds(base, CH)])
    return carry

  lax.fori_loop(0, npairs // CH, chunk, 0)


def _k1_body(d_ref, m_ref, wd_ref, wm_ref, zh_ref, z_ref):
  p = pl.program_id(0)
  is_d = (p % 10) < (ND // 1000)
  x = jnp.where(is_d, d_ref[...], m_ref[...])
  w = jnp.where(is_d, wd_ref[...], wm_ref[...])
  blk = jnp.dot(x, w, preferred_element_type=jnp.float32)
  zh_ref[...] = blk
  z_ref[...] = blk


def _node_transform(d_sim, m_sim, wd, wm):
  return pl.pallas_call(
      _k1_body,
      grid=(20,),
      in_specs=[
          pl.BlockSpec((1000, D), lambda g: (g % 10, 0)),
          pl.BlockSpec((1000, D), lambda g: (g % 10, 0)),
          pl.BlockSpec((D, HF), lambda g: (0, g // 10)),
          pl.BlockSpec((D, HF), lambda g: (0, g // 10)),
      ],
      out_specs=[
          pl.BlockSpec((1000, HF), lambda g: (g, 0)),
          pl.BlockSpec((1000, HF), lambda g: (g % 10, g // 10)),
      ],
      out_shape=(_f32(2 * N, HF), _f32(N, D)),
  )(d_sim, m_sim, wd, wm)


def _k2_body(mx_ref, dg_ref, c_ref, n_ref):
  c = jnp.max(mx_ref[...], axis=0)
  c_ref[...] = jnp.maximum(c, 0.0)
  deg = jnp.sum(dg_ref[...], axis=0)
  n_ref[...] = jnp.where(deg > 0, lax.rsqrt(jnp.maximum(deg, 1e-30)), 0.0)


def _combine_stats(maxpart, degpart):
  return pl.pallas_call(
      _k2_body,
      out_shape=(_f32(NB), _f32(NB)),
  )(maxpart, degpart)


def _k3_body(num_ref, den_ref, nrm_ref, ft_ref, g0_ref):
  i = pl.program_id(0)
  den = jnp.sum(den_ref[:, pl.ds(i % 8 * 1280, 1280)], axis=0)
  nrm = nrm_ref[pl.ds(i % 8 * 1280, 1280)]
  den = jnp.where(den > 0, den, 1.0)
  feats = _elu(num_ref[...] / den[:, None])
  ft_ref[...] = feats
  g0_ref[...] = feats * nrm[:, None]


def _feats_g0(num, den, normv):
  return pl.pallas_call(
      _k3_body,
      grid=(16,),
      in_specs=[
          pl.BlockSpec((1280, HF), lambda i: (i, 0)),
          pl.BlockSpec((NW, NB), lambda i: (0, 0)),
          pl.BlockSpec((NB,), lambda i: (0,)),
      ],
      out_specs=[
          pl.BlockSpec((1280, HF), lambda i: (i, 0)),
          pl.BlockSpec((1280, HF), lambda i: (i, 0)),
      ],
      out_shape=(_f32(NC * NB, HF), _f32(NC * NB, HF)),
  )(num.reshape(NC * NB, HF), den, normv)


def _k4_body(ft_ref, f1_ref, f2_ref, f3_ref, d_ref, m_ref, s_ref, wd_ref,
             bd_ref, wm_ref, bm_ref, h_ref):
  p = pl.program_id(0)
  hout = jnp.zeros((1000, F), jnp.float32)
  for ref in (ft_ref, f1_ref, f2_ref, f3_ref):
    hk = jnp.concatenate([ref[0], ref[1]], axis=1)
    sk = _sigmoid(jnp.dot(hk, s_ref[...], precision=lax.Precision.HIGHEST,
                          preferred_element_type=jnp.float32))
    hout = hout + sk[:, None] * hk
  is_d = p < (ND // 1000)
  sim = jnp.where(is_d, d_ref[...], m_ref[...])
  w = jnp.where(is_d, wd_ref[...], wm_ref[...])
  b = jnp.where(is_d, bd_ref[...], bm_ref[...])
  x = jnp.concatenate([hout, sim], axis=1)
  h_ref[...] = _elu(
      jnp.dot(x, w, precision=lax.Precision.HIGHEST,
              preferred_element_type=jnp.float32) + b[None, :])


def _final_mlp(ft, f1, f2, f3, d_sim, m_sim, s, wd_fc, bd_fc, wm_fc, bm_fc):
  fspec = pl.BlockSpec((NC, 1000, HF), lambda i: (0, i, 0))
  return pl.pallas_call(
      _k4_body,
      grid=(10,),
      in_specs=[
          fspec, fspec, fspec, fspec,
          pl.BlockSpec((1000, D), lambda i: (i, 0)),
          pl.BlockSpec((1000, D), lambda i: (i, 0)),
          pl.BlockSpec((F,), lambda i: (0,)),
          pl.BlockSpec((F + D, OUT), lambda i: (0, 0)),
          pl.BlockSpec((OUT,), lambda i: (0,)),
          pl.BlockSpec((F + D, OUT), lambda i: (0, 0)),
          pl.BlockSpec((OUT,), lambda i: (0,)),
      ],
      out_specs=pl.BlockSpec((1000, OUT), lambda i: (i, 0)),
      out_shape=_f32(N, OUT),
  )(ft, f1, f2, f3, d_sim, m_sim, s, wd_fc, bd_fc, wm_fc, bm_fc)


def kernel(d_sim, m_sim, W_d1, W_m1, W_d2, W_m2, s, Wd_fc, bd_fc, Wm_fc,
           bm_fc, Wp, bp, edge_index, diseases, mirnas):
  src = edge_index[0]
  dst = edge_index[1]
  pad = EPAD - E
  src_pad = jnp.concatenate([src, jnp.zeros((pad,), jnp.int32)])
  dst_pad = jnp.concatenate([dst, jnp.full((pad,), N, jnp.int32)])
  dstg = jnp.minimum(dst_pad, N - 1)
  srcA3 = src_pad.reshape(NW, NCHA, CHA)
  dgA3 = dstg.reshape(NW, NCHA, CHA)
  dstA3 = dst_pad.reshape(NW, NCHA, CHA)
  srczB = jnp.stack([src_pad, src_pad + N]).reshape(NC, NS, NCHB, CHB)
  dstB3 = dst_pad.reshape(NS, NCHB, CHB)
  dstE3 = dst_pad.reshape(NW, NCHA, CHA)
  srcgC = jnp.stack([src_pad, src_pad + NB]).reshape(NC, NS, NCHC, CHC)
  dstC3 = dst_pad.reshape(NS, NCHC, CHC)
  zeros_nb = jnp.zeros((NB,), jnp.float32)
  zeros_rows = jnp.zeros((NSLC, HF), jnp.float32)
  wpd = Wp[:OUT, 0]
  wpm = Wp[OUT:, 0]
  bp16 = jnp.full((L,), 0.0, jnp.float32) + bp[0]

  zh, z = _node_transform(d_sim, m_sim, W_d2, W_m2)
  e_pad, maxpart, degpart = _edge_scores(z, srcA3, dgA3, dstA3, zeros_nb)
  cvec, normv = _combine_stats(maxpart, degpart)
  exfull, denpart = _edge_weights(e_pad, dstE3, cvec, zeros_nb)
  num = _attn_aggregate(zh, srczB, dstB3,
                        exfull.reshape(NS, NCHB, CHB), zeros_rows)
  feats, g0 = _feats_g0(num, denpart, normv)
  f_all, _, _ = _dagnn(g0, srcgC, dstC3, normv, zeros_rows)
  ftr = feats.reshape(NC, NB, HF)
  f1 = f_all[0].reshape(NC, NB, HF)
  f2 = f_all[1].reshape(NC, NB, HF)
  f3 = f_all[2].reshape(NC, NB, HF)
  h = _final_mlp(ftr, f1, f2, f3, d_sim, m_sim, s[:, 0], Wd_fc, bd_fc,
                 Wm_fc, bm_fc)
  o = _pair_scores(h, diseases, mirnas, wpd, wpm, bp16)
  return o.reshape(BP, 1)


# speedup vs baseline: 3.0942x; 1.0136x over previous
import functools

import jax
import jax.numpy as jnp
from jax import lax
from jax.experimental import pallas as pl
from jax.experimental.pallas import tpu as pltpu
from jax.experimental.pallas import tpu_sc as plsc

N = 10000
ND = 4000
E = 160000
D = 256
F = 256
OUT = 128
KHOP = 3
BP = 16384
SLOPE = 0.2

NC = 2
NS = 16
L = 16
NW = NC * NS

EPAD = 163840
EWA = EPAD // NW
EWS = EPAD // NS
CH = 64
CHA = 64
NCHA = EWA // CHA
CHB = 128
NCHB = EWS // CHB
CHC = 128
NCHC = EWS // CHC
NB = 10240
NSLC = NB // NS
HF = 128

_mesh = plsc.VectorSubcoreMesh(
    core_axis_name="c", subcore_axis_name="s", num_cores=NC, num_subcores=NS)


def _elu(x):
  return jnp.where(x > 0, x, jnp.exp(jnp.minimum(x, 0.0)) - 1.0)


def _sigmoid(x):
  return 1.0 / (1.0 + jnp.exp(-x))


def _f32(*shape):
  return jax.ShapeDtypeStruct(shape, jnp.float32)


def _wid():
  c = lax.axis_index("c")
  s = lax.axis_index("s")
  return c, s, c * NS + s


def _iota16():
  return lax.broadcasted_iota(jnp.int32, (L,), 0)


def _vtake(x, idx):
  return x.at[idx].get(mode="promise_in_bounds")


def _runs(ks, vs, combine):
  iota = _iota16()
  for sh in (1, 2, 4, 8):
    pidx = jnp.maximum(iota - sh, 0)
    kp = _vtake(ks, pidx)
    vp = _vtake(vs, pidx)
    vs = jnp.where((kp == ks) & (iota >= sh), combine(vs, vp), vs)
  knext = _vtake(ks, jnp.minimum(iota + 1, L - 1))
  is_last = (ks != knext) | (iota == L - 1)
  return vs, is_last


@functools.partial(
    pl.kernel,
    out_type=(_f32(EPAD), _f32(NW, NB), _f32(NW, NB)),
    mesh=_mesh,
    compiler_params=pltpu.CompilerParams(needs_layout_passes=False),
    scratch_types=[
        pltpu.VMEM((NCHA, CHA), jnp.int32),
        pltpu.VMEM((NCHA, CHA), jnp.int32),
        pltpu.VMEM((NCHA, CHA), jnp.int32),
        pltpu.VMEM((EWA,), jnp.float32),
        pltpu.VMEM((NB,), jnp.float32),
        pltpu.VMEM((NB,), jnp.float32),
        pltpu.VMEM((CHA, D), jnp.float32),
        pltpu.VMEM((CHA, D), jnp.float32),
        pltpu.VMEM((CHA, D), jnp.float32),
        pltpu.VMEM((CHA, D), jnp.float32),
        pltpu.SemaphoreType.DMA,
        pltpu.SemaphoreType.DMA,
    ],
)
def _edge_scores(z, srcA, dgA, dstA, zeros_nb, e_out, maxpart, degpart,
                 sA, gA, kA, ebuf, mbins, dbins, sr0, dr0, sr1, dr1, g0, g1):
  cid, sid, wid = _wid()
  pltpu.sync_copy(srcA.at[wid], sA)
  pltpu.sync_copy(dgA.at[wid], gA)
  pltpu.sync_copy(dstA.at[wid], kA)
  pltpu.sync_copy(zeros_nb, mbins)
  pltpu.sync_copy(zeros_nb, dbins)
  iota = _iota16()
  ones = jnp.ones((L,), jnp.float32)

  def process(k, sr, dr):
    def grp(g, carry2):
      def edge(j, ev):
        row = g * L + j
        acc = jnp.zeros((L,), jnp.float32)
        for m in range(D // L):
          sl2 = pl.ds(m * L, L)
          acc = acc + sr[row, sl2] * dr[row, sl2]
        dot = jnp.sum(acc)
        e = jnp.where(dot > 0, dot, SLOPE * dot)
        return jnp.where(iota == j, e, ev)

      ev = lax.fori_loop(0, L, edge, jnp.zeros((L,), jnp.float32))
      ebuf[pl.ds(k * CHA + g * L, L)] = ev
      ks, vs = plsc.sort_key_val(kA[k, pl.ds(g * L, L)], ev)
      vmax, is_last = _runs(ks, vs, jnp.maximum)
      cur = plsc.load_gather(mbins, [ks])
      plsc.store_scatter(mbins, [ks], jnp.maximum(cur, vmax), mask=is_last)
      cnt, _ = _runs(ks, ones, lambda a, b: a + b)
      dcur = plsc.load_gather(dbins, [ks])
      plsc.store_scatter(dbins, [ks], dcur + cnt, mask=is_last)
      return carry2

    lax.fori_loop(0, CHA // L, grp, 0)

  def pair(kk, carry):
    k0 = 2 * kk
    k1 = k0 + 1
    ds0 = pltpu.async_copy(z.at[sA.at[k0]], sr0, g0)
    dd0 = pltpu.async_copy(z.at[gA.at[k0]], dr0, g0)
    ds1 = pltpu.async_copy(z.at[sA.at[k1]], sr1, g1)
    dd1 = pltpu.async_copy(z.at[gA.at[k1]], dr1, g1)
    ds0.wait()
    dd0.wait()
    process(k0, sr0, dr0)
    ds1.wait()
    dd1.wait()
    process(k1, sr1, dr1)
    return carry

  lax.fori_loop(0, NCHA // 2, pair, 0)
  pltpu.sync_copy(ebuf, e_out.at[pl.ds(wid * EWA, EWA)])
  pltpu.sync_copy(mbins, maxpart.at[wid])
  pltpu.sync_copy(dbins, degpart.at[wid])


@functools.partial(
    pl.kernel,
    out_type=(_f32(EPAD), _f32(NW, NB)),
    mesh=_mesh,
    compiler_params=pltpu.CompilerParams(needs_layout_passes=False),
    scratch_types=[
        pltpu.VMEM((NCHA, CHA), jnp.int32),
        pltpu.VMEM((EWA,), jnp.float32),
        pltpu.VMEM((EWA,), jnp.float32),
        pltpu.VMEM((NB,), jnp.float32),
        pltpu.VMEM((NB,), jnp.float32),
    ],
)
def _edge_weights(e_in, dstA, cvec, zeros_nb, ex_out, denpart, kA, e1, exb,
                  cbins, dbins):
  cid, sid, wid = _wid()
  pltpu.sync_copy(dstA.at[wid], kA)
  pltpu.sync_copy(e_in.at[pl.ds(wid * EWA, EWA)], e1)
  pltpu.sync_copy(cvec, cbins)
  pltpu.sync_copy(zeros_nb, dbins)

  def chunk(k, carry):
    def grp(g, carry2):
      dk = kA[k, pl.ds(g * L, L)]
      evv = e1[pl.ds(k * CHA + g * L, L)]
      cg = plsc.load_gather(cbins, [dk])
      exv = jnp.exp(evv - cg)
      exb[pl.ds(k * CHA + g * L, L)] = exv
      ks, xs = plsc.sort_key_val(dk, exv)
      ssum, is_last = _runs(ks, xs, lambda a, b: a + b)
      cur = plsc.load_gather(dbins, [ks])
      plsc.store_scatter(dbins, [ks], cur + ssum, mask=is_last)
      return carry2

    return lax.fori_loop(0, CHA // L, grp, carry)

  lax.fori_loop(0, NCHA, chunk, 0)
  pltpu.sync_copy(exb, ex_out.at[pl.ds(wid * EWA, EWA)])
  pltpu.sync_copy(dbins, denpart.at[wid])


@functools.partial(
    pl.kernel,
    out_type=_f32(NC, NB, HF),
    mesh=_mesh,
    compiler_params=pltpu.CompilerParams(needs_layout_passes=False),
    scratch_types=[
        pltpu.VMEM((CHB,), jnp.int32),
        pltpu.VMEM((CHB,), jnp.int32),
        pltpu.VMEM((CHB,), jnp.int32),
        pltpu.VMEM((CHB,), jnp.int32),
        pltpu.VMEM((CHB,), jnp.float32),
        pltpu.VMEM((CHB,), jnp.float32),
        pltpu.VMEM((CHB, HF), jnp.float32),
        pltpu.VMEM((CHB, HF), jnp.float32),
        pltpu.VMEM_SHARED((NB, HF), jnp.float32),
        pltpu.SemaphoreType.DMA,
        pltpu.SemaphoreType.DMA,
        pltpu.SemaphoreType.DMA,
        pltpu.SemaphoreType.DMA,
    ],
)
def _attn_aggregate(zh, srczB, dstB, exB, zeros_rows, num_out, si0, si1,
                    di0, di1, ex0, ex1, r0, r1, numsh, g0, g1, s0, s1):
  cid, sid, wid = _wid()
  pltpu.sync_copy(zeros_rows, numsh.at[pl.ds(sid * NSLC, NSLC)])
  plsc.subcore_barrier()

  def ldidx(k, si, di, exb):
    pltpu.sync_copy(srczB.at[cid, sid, k], si)
    pltpu.sync_copy(dstB.at[sid, k], di)
    pltpu.sync_copy(exB.at[sid, k], exb)

  def scale(r, exb):
    def grp(g, carry2):
      exv = exb[pl.ds(g * L, L)]

      def edge(j, carry3):
        row = g * L + j
        sv = _vtake(exv, jnp.full((L,), j, jnp.int32))
        for m in range(HF // L):
          sl2 = pl.ds(m * L, L)
          r[row, sl2] = r[row, sl2] * sv
        return carry3

      return lax.fori_loop(0, L, edge, carry2)

    lax.fori_loop(0, CHB // L, grp, 0)

  def pair(kk, carry):
    k0 = 2 * kk
    k1 = k0 + 1
    ldidx(k0, si0, di0, ex0)
    dg0 = pltpu.async_copy(zh.at[si0], r0, g0)
    ldidx(k1, si1, di1, ex1)
    dg1 = pltpu.async_copy(zh.at[si1], r1, g1)
    dg0.wait()
    scale(r0, ex0)
    d0 = pltpu.async_copy(r0, numsh.at[di0], s0, add=True)
    dg1.wait()
    scale(r1, ex1)
    d1 = pltpu.async_copy(r1, numsh.at[di1], s1, add=True)
    d0.wait()
    d1.wait()
    return carry

  lax.fori_loop(0, NCHB // 2, pair, 0)
  plsc.subcore_barrier()
  pltpu.sync_copy(numsh.at[pl.ds(sid * NSLC, NSLC)],
                  num_out.at[cid, pl.ds(sid * NSLC, NSLC)])


@functools.partial(
    pl.kernel,
    out_type=(_f32(KHOP, NC * NB, HF), _f32(NC * NB, HF), _f32(NC * NB, HF)),
    mesh=_mesh,
    compiler_params=pltpu.CompilerParams(needs_layout_passes=False),
    scratch_types=[
        pltpu.VMEM((CHC,), jnp.int32),
        pltpu.VMEM((CHC,), jnp.int32),
        pltpu.VMEM((CHC,), jnp.int32),
        pltpu.VMEM((CHC,), jnp.int32),
        pltpu.VMEM((NSLC,), jnp.float32),
        pltpu.VMEM((CHC, HF), jnp.float32),
        pltpu.VMEM((CHC, HF), jnp.float32),
        pltpu.VMEM_SHARED((NB, HF), jnp.float32),
        pltpu.SemaphoreType.DMA,
        pltpu.SemaphoreType.DMA,
        pltpu.SemaphoreType.DMA,
        pltpu.SemaphoreType.DMA,
        pltpu.SemaphoreType.DMA,
        pltpu.SemaphoreType.DMA,
    ],
)
def _dagnn(g0t, srcgC, dstC, normv, zeros_rows, f_all, gta, gtb, si0, si1,
           di0, di1, normb, r0, r1, binsh, g0, g1, s0, s1, i0, i1):
  cid, sid, _ = _wid()
  pltpu.sync_copy(normv.at[pl.ds(sid * NSLC, NSLC)], normb)
  tabs = (g0t, gta, gtb)
  for r in range(KHOP):
    gtab = tabs[r]
    pltpu.sync_copy(zeros_rows, binsh.at[pl.ds(sid * NSLC, NSLC)])
    plsc.subcore_barrier()

    def pair(kk, carry, gtab=gtab):
      k0 = 2 * kk
      k1 = k0 + 1
      a0 = pltpu.async_copy(srcgC.at[cid, sid, k0], si0, i0)
      b0 = pltpu.async_copy(dstC.at[sid, k0], di0, i0)
      a1 = pltpu.async_copy(srcgC.at[cid, sid, k1], si1, i1)
      b1 = pltpu.async_copy(dstC.at[sid, k1], di1, i1)
      a0.wait()
      b0.wait()
      dg0 = pltpu.async_copy(gtab.at[si0], r0, g0)
      a1.wait()
      b1.wait()
      dg1 = pltpu.async_copy(gtab.at[si1], r1, g1)
      dg0.wait()
      d0 = pltpu.async_copy(r0, binsh.at[di0], s0, add=True)
      dg1.wait()
      d1 = pltpu.async_copy(r1, binsh.at[di1], s1, add=True)
      d0.wait()
      d1.wait()
      return carry

    lax.fori_loop(0, NCHC // 2, pair, 0)
    plsc.subcore_barrier()

    def slice_k(kk, carry):
      off = sid * NSLC + kk * CHC
      pltpu.sync_copy(binsh.at[pl.ds(off, CHC)], r0)

      def grp(g, carry2):
        nv = normb[pl.ds(kk * CHC + g * L, L)]

        def node(j, carry3):
          row = g * L + j
          sv = _vtake(nv, jnp.full((L,), j, jnp.int32))
          for m in range(HF // L):
            sl = pl.ds(m * L, L)
            v = r0[row, sl] * sv
            r1[row, sl] = v
            r0[row, sl] = v * sv
          return carry3

        return lax.fori_loop(0, L, node, carry2)

      lax.fori_loop(0, CHC // L, grp, 0)
      pltpu.sync_copy(r1, f_all.at[r, pl.ds(cid * NB + off, CHC)])
      if r == 0:
        pltpu.sync_copy(r0, gta.at[pl.ds(cid * NB + off, CHC)])
      elif r == 1:
        pltpu.sync_copy(r0, gtb.at[pl.ds(cid * NB + off, CHC)])
      return carry

    lax.fori_loop(0, NSLC // CHC, slice_k, 0)
    plsc.subcore_barrier()


@functools.partial(
    pl.kernel,
    out_type=_f32(BP),
    mesh=_mesh,
    compiler_params=pltpu.CompilerParams(needs_layout_passes=False),
    scratch_types=[
        pltpu.VMEM((CH,), jnp.int32),
        pltpu.VMEM((CH,), jnp.int32),
        pltpu.VMEM((CH, OUT), jnp.float32),
        pltpu.VMEM((CH, OUT), jnp.float32),
        pltpu.VMEM((OUT,), jnp.float32),
        pltpu.VMEM((OUT,), jnp.float32),
        pltpu.VMEM((L,), jnp.float32),
        pltpu.VMEM((CH,), jnp.float32),
        pltpu.SemaphoreType.DMA,
    ],
)
def _pair_scores(h, dis, mir, wpd, wpm, bp16, o_out, dib, mib, hrd, hrm,
                 wdb, wmb, bpb, obuf, sem):
  _, _, wid = _wid()
  pltpu.sync_copy(wpd, wdb)
  pltpu.sync_copy(wpm, wmb)
  pltpu.sync_copy(bp16, bpb)
  npairs = BP // NW

  def chunk(i, carry):
    base = wid * npairs + i * CH
    pltpu.sync_copy(dis.at[pl.ds(base, CH)], dib)
    pltpu.sync_copy(mir.at[pl.ds(base, CH)], mib)
    c1 = pltpu.async_copy(h.at[dib], hrd, sem)
    c2 = pltpu.async_copy(h.at[mib], hrm, sem)
    c1.wait()
    c2.wait()

    iota = _iota16()
    for g in range(CH // L):

      def pair(j, tv, _g=g):
        row = _g * L + j
        acc = jnp.zeros((L,), jnp.float32)
        for m in range(OUT // L):
          sl2 = pl.ds(m * L, L)
          acc = acc + hrd[row, sl2] * wdb[sl2]
          acc = acc + hrm[row, sl2] * wmb[sl2]
        return jnp.where(iota == j, jnp.sum(acc), tv)

      tv = lax.fori_loop(0, L, pair, jnp.zeros((L,), jnp.float32))
      t = tv + bpb[...]
      obuf[pl.ds(g * L, L)] = 1.0 / (1.0 + jnp.exp(-t))
    pltpu.sync_copy(obuf, o_out.at[pl.ds(base, CH)])
    return carry

  lax.fori_loop(0, npairs // CH, chunk, 0)


def _k1_body(d_ref, m_ref, wd_ref, wm_ref, zh_ref, z_ref):
  p = pl.program_id(0)
  is_d = (p % 10) < (ND // 1000)
  x = jnp.where(is_d, d_ref[...], m_ref[...])
  w = jnp.where(is_d, wd_ref[...], wm_ref[...])
  blk = jnp.dot(x, w, preferred_element_type=jnp.float32)
  zh_ref[...] = blk
  z_ref[...] = blk


def _node_transform(d_sim, m_sim, wd, wm):
  return pl.pallas_call(
      _k1_body,
      grid=(20,),
      in_specs=[
          pl.BlockSpec((1000, D), lambda g: (g % 10, 0)),
          pl.BlockSpec((1000, D), lambda g: (g % 10, 0)),
          pl.BlockSpec((D, HF), lambda g: (0, g // 10)),
          pl.BlockSpec((D, HF), lambda g: (0, g // 10)),
      ],
      out_specs=[
          pl.BlockSpec((1000, HF), lambda g: (g, 0)),
          pl.BlockSpec((1000, HF), lambda g: (g % 10, g // 10)),
      ],
      out_shape=(_f32(2 * N, HF), _f32(N, D)),
  )(d_sim, m_sim, wd, wm)


def _k2_body(mx_ref, dg_ref, c_ref, n_ref):
  c = jnp.max(mx_ref[...], axis=0)
  c_ref[...] = jnp.maximum(c, 0.0)
  deg = jnp.sum(dg_ref[...], axis=0)
  n_ref[...] = jnp.where(deg > 0, lax.rsqrt(jnp.maximum(deg, 1e-30)), 0.0)


def _combine_stats(maxpart, degpart):
  return pl.pallas_call(
      _k2_body,
      out_shape=(_f32(NB), _f32(NB)),
  )(maxpart, degpart)


def _k3_body(num_ref, den_ref, nrm_ref, ft_ref, g0_ref):
  i = pl.program_id(0)
  den = jnp.sum(den_ref[:, pl.ds(i % 8 * 1280, 1280)], axis=0)
  nrm = nrm_ref[pl.ds(i % 8 * 1280, 1280)]
  den = jnp.where(den > 0, den, 1.0)
  feats = _elu(num_ref[...] / den[:, None])
  ft_ref[...] = feats
  g0_ref[...] = feats * nrm[:, None]


def _feats_g0(num, den, normv):
  return pl.pallas_call(
      _k3_body,
      grid=(16,),
      in_specs=[
          pl.BlockSpec((1280, HF), lambda i: (i, 0)),
          pl.BlockSpec((NW, NB), lambda i: (0, 0)),
          pl.BlockSpec((NB,), lambda i: (0,)),
      ],
      out_specs=[
          pl.BlockSpec((1280, HF), lambda i: (i, 0)),
          pl.BlockSpec((1280, HF), lambda i: (i, 0)),
      ],
      out_shape=(_f32(NC * NB, HF), _f32(NC * NB, HF)),
  )(num.reshape(NC * NB, HF), den, normv)


def _k4_body(ft_ref, f1_ref, f2_ref, f3_ref, d_ref, m_ref, s_ref, wd_ref,
             bd_ref, wm_ref, bm_ref, h_ref):
  p = pl.program_id(0)
  hout = jnp.zeros((1000, F), jnp.float32)
  for ref in (ft_ref, f1_ref, f2_ref, f3_ref):
    hk = jnp.concatenate([ref[0], ref[1]], axis=1)
    sk = _sigmoid(jnp.dot(hk, s_ref[...], precision=lax.Precision.HIGHEST,
                          preferred_element_type=jnp.float32))
    hout = hout + sk[:, None] * hk
  is_d = p < (ND // 1000)
  sim = jnp.where(is_d, d_ref[...], m_ref[...])
  w = jnp.where(is_d, wd_ref[...], wm_ref[...])
  b = jnp.where(is_d, bd_ref[...], bm_ref[...])
  x = jnp.concatenate([hout, sim], axis=1)
  h_ref[...] = _elu(
      jnp.dot(x, w, precision=lax.Precision.HIGHEST,
              preferred_element_type=jnp.float32) + b[None, :])


def _final_mlp(ft, f1, f2, f3, d_sim, m_sim, s, wd_fc, bd_fc, wm_fc, bm_fc):
  fspec = pl.BlockSpec((NC, 1000, HF), lambda i: (0, i, 0))
  return pl.pallas_call(
      _k4_body,
      grid=(10,),
      in_specs=[
          fspec, fspec, fspec, fspec,
          pl.BlockSpec((1000, D), lambda i: (i, 0)),
          pl.BlockSpec((1000, D), lambda i: (i, 0)),
          pl.BlockSpec((F,), lambda i: (0,)),
          pl.BlockSpec((F + D, OUT), lambda i: (0, 0)),
          pl.BlockSpec((OUT,), lambda i: (0,)),
          pl.BlockSpec((F + D, OUT), lambda i: (0, 0)),
          pl.BlockSpec((OUT,), lambda i: (0,)),
      ],
      out_specs=pl.BlockSpec((1000, OUT), lambda i: (i, 0)),
      out_shape=_f32(N, OUT),
  )(ft, f1, f2, f3, d_sim, m_sim, s, wd_fc, bd_fc, wm_fc, bm_fc)


def kernel(d_sim, m_sim, W_d1, W_m1, W_d2, W_m2, s, Wd_fc, bd_fc, Wm_fc,
           bm_fc, Wp, bp, edge_index, diseases, mirnas):
  src = edge_index[0]
  dst = edge_index[1]
  pad = EPAD - E
  src_pad = jnp.concatenate([src, jnp.zeros((pad,), jnp.int32)])
  dst_pad = jnp.concatenate([dst, jnp.full((pad,), N, jnp.int32)])
  dstg = jnp.minimum(dst_pad, N - 1)
  srcA3 = src_pad.reshape(NW, NCHA, CHA)
  dgA3 = dstg.reshape(NW, NCHA, CHA)
  dstA3 = dst_pad.reshape(NW, NCHA, CHA)
  srczB = jnp.stack([src_pad, src_pad + N]).reshape(NC, NS, NCHB, CHB)
  dstB3 = dst_pad.reshape(NS, NCHB, CHB)
  dstE3 = dst_pad.reshape(NW, NCHA, CHA)
  srcgC = jnp.stack([src_pad, src_pad + NB]).reshape(NC, NS, NCHC, CHC)
  dstC3 = dst_pad.reshape(NS, NCHC, CHC)
  zeros_nb = jnp.zeros((NB,), jnp.float32)
  zeros_rows = jnp.zeros((NSLC, HF), jnp.float32)
  wpd = Wp[:OUT, 0]
  wpm = Wp[OUT:, 0]
  bp16 = jnp.full((L,), 0.0, jnp.float32) + bp[0]

  zh, z = _node_transform(d_sim, m_sim, W_d2, W_m2)
  e_pad, maxpart, degpart = _edge_scores(z, srcA3, dgA3, dstA3, zeros_nb)
  cvec, normv = _combine_stats(maxpart, degpart)
  exfull, denpart = _edge_weights(e_pad, dstE3, cvec, zeros_nb)
  num = _attn_aggregate(zh, srczB, dstB3,
                        exfull.reshape(NS, NCHB, CHB), zeros_rows)
  feats, g0 = _feats_g0(num, denpart, normv)
  f_all, _, _ = _dagnn(g0, srcgC, dstC3, normv, zeros_rows)
  ftr = feats.reshape(NC, NB, HF)
  f1 = f_all[0].reshape(NC, NB, HF)
  f2 = f_all[1].reshape(NC, NB, HF)
  f3 = f_all[2].reshape(NC, NB, HF)
  h = _final_mlp(ftr, f1, f2, f3, d_sim, m_sim, s[:, 0], Wd_fc, bd_fc,
                 Wm_fc, bm_fc)
  o = _pair_scores(h, diseases, mirnas, wpd, wpm, bp16)
  return o.reshape(BP, 1)


# speedup vs baseline: 3.1640x; 1.0226x over previous
import functools

import jax
import jax.numpy as jnp
from jax import lax
from jax.experimental import pallas as pl
from jax.experimental.pallas import tpu as pltpu
from jax.experimental.pallas import tpu_sc as plsc

N = 10000
ND = 4000
E = 160000
D = 256
F = 256
OUT = 128
KHOP = 3
BP = 16384
SLOPE = 0.2

NC = 2
NS = 16
L = 16
NW = NC * NS

EPAD = 163840
EWA = EPAD // NW
EWS = EPAD // NS
CH = 64
CHA = 64
NCHA = EWA // CHA
CHB = 128
NCHB = EWS // CHB
CHC = 64
NCHC = EWS // CHC
NB = 10240
NSLC = NB // NS
HF = 128

_mesh = plsc.VectorSubcoreMesh(
    core_axis_name="c", subcore_axis_name="s", num_cores=NC, num_subcores=NS)


def _elu(x):
  return jnp.where(x > 0, x, jnp.exp(jnp.minimum(x, 0.0)) - 1.0)


def _sigmoid(x):
  return 1.0 / (1.0 + jnp.exp(-x))


def _f32(*shape):
  return jax.ShapeDtypeStruct(shape, jnp.float32)


def _wid():
  c = lax.axis_index("c")
  s = lax.axis_index("s")
  return c, s, c * NS + s


def _iota16():
  return lax.broadcasted_iota(jnp.int32, (L,), 0)


def _vtake(x, idx):
  return x.at[idx].get(mode="promise_in_bounds")


def _runs(ks, vs, combine):
  iota = _iota16()
  for sh in (1, 2, 4, 8):
    pidx = jnp.maximum(iota - sh, 0)
    kp = _vtake(ks, pidx)
    vp = _vtake(vs, pidx)
    vs = jnp.where((kp == ks) & (iota >= sh), combine(vs, vp), vs)
  knext = _vtake(ks, jnp.minimum(iota + 1, L - 1))
  is_last = (ks != knext) | (iota == L - 1)
  return vs, is_last


@functools.partial(
    pl.kernel,
    out_type=(_f32(EPAD), _f32(NW, NB), _f32(NW, NB)),
    mesh=_mesh,
    compiler_params=pltpu.CompilerParams(needs_layout_passes=False),
    scratch_types=[
        pltpu.VMEM((NCHA, CHA), jnp.int32),
        pltpu.VMEM((NCHA, CHA), jnp.int32),
        pltpu.VMEM((NCHA, CHA), jnp.int32),
        pltpu.VMEM((EWA,), jnp.float32),
        pltpu.VMEM((NB,), jnp.float32),
        pltpu.VMEM((NB,), jnp.float32),
        pltpu.VMEM((CHA, D), jnp.float32),
        pltpu.VMEM((CHA, D), jnp.float32),
        pltpu.VMEM((CHA, D), jnp.float32),
        pltpu.VMEM((CHA, D), jnp.float32),
        pltpu.SemaphoreType.DMA,
        pltpu.SemaphoreType.DMA,
    ],
)
def _edge_scores(z, srcA, dgA, dstA, zeros_nb, e_out, maxpart, degpart,
                 sA, gA, kA, ebuf, mbins, dbins, sr0, dr0, sr1, dr1, g0, g1):
  cid, sid, wid = _wid()
  pltpu.sync_copy(srcA.at[wid], sA)
  pltpu.sync_copy(dgA.at[wid], gA)
  pltpu.sync_copy(dstA.at[wid], kA)
  pltpu.sync_copy(zeros_nb, mbins)
  pltpu.sync_copy(zeros_nb, dbins)
  iota = _iota16()
  ones = jnp.ones((L,), jnp.float32)

  def process(k, sr, dr):
    def grp(g, carry2):
      def edge(j, ev):
        row = g * L + j
        acc = jnp.zeros((L,), jnp.float32)
        for m in range(D // L):
          sl2 = pl.ds(m * L, L)
          acc = acc + sr[row, sl2] * dr[row, sl2]
        dot = jnp.sum(acc)
        e = jnp.where(dot > 0, dot, SLOPE * dot)
        return jnp.where(iota == j, e, ev)

      ev = lax.fori_loop(0, L, edge, jnp.zeros((L,), jnp.float32))
      ebuf[pl.ds(k * CHA + g * L, L)] = ev
      ks, vs = plsc.sort_key_val(kA[k, pl.ds(g * L, L)], ev)
      vmax, is_last = _runs(ks, vs, jnp.maximum)
      cur = plsc.load_gather(mbins, [ks])
      plsc.store_scatter(mbins, [ks], jnp.maximum(cur, vmax), mask=is_last)
      cnt, _ = _runs(ks, ones, lambda a, b: a + b)
      dcur = plsc.load_gather(dbins, [ks])
      plsc.store_scatter(dbins, [ks], dcur + cnt, mask=is_last)
      return carry2

    lax.fori_loop(0, CHA // L, grp, 0)

  def pair(kk, carry):
    k0 = 2 * kk
    k1 = k0 + 1
    ds0 = pltpu.async_copy(z.at[sA.at[k0]], sr0, g0)
    dd0 = pltpu.async_copy(z.at[gA.at[k0]], dr0, g0)
    ds1 = pltpu.async_copy(z.at[sA.at[k1]], sr1, g1)
    dd1 = pltpu.async_copy(z.at[gA.at[k1]], dr1, g1)
    ds0.wait()
    dd0.wait()
    process(k0, sr0, dr0)
    ds1.wait()
    dd1.wait()
    process(k1, sr1, dr1)
    return carry

  lax.fori_loop(0, NCHA // 2, pair, 0)
  pltpu.sync_copy(ebuf, e_out.at[pl.ds(wid * EWA, EWA)])
  pltpu.sync_copy(mbins, maxpart.at[wid])
  pltpu.sync_copy(dbins, degpart.at[wid])


@functools.partial(
    pl.kernel,
    out_type=(_f32(EPAD), _f32(NW, NB)),
    mesh=_mesh,
    compiler_params=pltpu.CompilerParams(needs_layout_passes=False),
    scratch_types=[
        pltpu.VMEM((NCHA, CHA), jnp.int32),
        pltpu.VMEM((EWA,), jnp.float32),
        pltpu.VMEM((EWA,), jnp.float32),
        pltpu.VMEM((NB,), jnp.float32),
        pltpu.VMEM((NB,), jnp.float32),
    ],
)
def _edge_weights(e_in, dstA, cvec, zeros_nb, ex_out, denpart, kA, e1, exb,
                  cbins, dbins):
  cid, sid, wid = _wid()
  pltpu.sync_copy(dstA.at[wid], kA)
  pltpu.sync_copy(e_in.at[pl.ds(wid * EWA, EWA)], e1)
  pltpu.sync_copy(cvec, cbins)
  pltpu.sync_copy(zeros_nb, dbins)

  def chunk(k, carry):
    def grp(g, carry2):
      dk = kA[k, pl.ds(g * L, L)]
      evv = e1[pl.ds(k * CHA + g * L, L)]
      cg = plsc.load_gather(cbins, [dk])
      exv = jnp.exp(evv - cg)
      exb[pl.ds(k * CHA + g * L, L)] = exv
      ks, xs = plsc.sort_key_val(dk, exv)
      ssum, is_last = _runs(ks, xs, lambda a, b: a + b)
      cur = plsc.load_gather(dbins, [ks])
      plsc.store_scatter(dbins, [ks], cur + ssum, mask=is_last)
      return carry2

    return lax.fori_loop(0, CHA // L, grp, carry)

  lax.fori_loop(0, NCHA, chunk, 0)
  pltpu.sync_copy(exb, ex_out.at[pl.ds(wid * EWA, EWA)])
  pltpu.sync_copy(dbins, denpart.at[wid])


@functools.partial(
    pl.kernel,
    out_type=_f32(NC, NB, HF),
    mesh=_mesh,
    compiler_params=pltpu.CompilerParams(needs_layout_passes=False),
    scratch_types=[
        pltpu.VMEM((CHB,), jnp.int32),
        pltpu.VMEM((CHB,), jnp.int32),
        pltpu.VMEM((CHB,), jnp.int32),
        pltpu.VMEM((CHB,), jnp.int32),
        pltpu.VMEM((CHB,), jnp.float32),
        pltpu.VMEM((CHB,), jnp.float32),
        pltpu.VMEM((CHB, HF), jnp.float32),
        pltpu.VMEM((CHB, HF), jnp.float32),
        pltpu.VMEM_SHARED((NB, HF), jnp.float32),
        pltpu.SemaphoreType.DMA,
        pltpu.SemaphoreType.DMA,
        pltpu.SemaphoreType.DMA,
        pltpu.SemaphoreType.DMA,
    ],
)
def _attn_aggregate(zh, srczB, dstB, exB, zeros_rows, num_out, si0, si1,
                    di0, di1, ex0, ex1, r0, r1, numsh, g0, g1, s0, s1):
  cid, sid, wid = _wid()
  pltpu.sync_copy(zeros_rows, numsh.at[pl.ds(sid * NSLC, NSLC)])
  plsc.subcore_barrier()

  def ldidx(k, si, di, exb):
    pltpu.sync_copy(srczB.at[cid, sid, k], si)
    pltpu.sync_copy(dstB.at[sid, k], di)
    pltpu.sync_copy(exB.at[sid, k], exb)

  def scale(r, exb):
    def grp(g, carry2):
      exv = exb[pl.ds(g * L, L)]

      def edge(j, carry3):
        row = g * L + j
        sv = _vtake(exv, jnp.full((L,), j, jnp.int32))
        for m in range(HF // L):
          sl2 = pl.ds(m * L, L)
          r[row, sl2] = r[row, sl2] * sv
        return carry3

      return lax.fori_loop(0, L, edge, carry2)

    lax.fori_loop(0, CHB // L, grp, 0)

  def pair(kk, carry):
    k0 = 2 * kk
    k1 = k0 + 1
    ldidx(k0, si0, di0, ex0)
    dg0 = pltpu.async_copy(zh.at[si0], r0, g0)
    ldidx(k1, si1, di1, ex1)
    dg1 = pltpu.async_copy(zh.at[si1], r1, g1)
    dg0.wait()
    scale(r0, ex0)
    d0 = pltpu.async_copy(r0, numsh.at[di0], s0, add=True)
    dg1.wait()
    scale(r1, ex1)
    d1 = pltpu.async_copy(r1, numsh.at[di1], s1, add=True)
    d0.wait()
    d1.wait()
    return carry

  lax.fori_loop(0, NCHB // 2, pair, 0)
  plsc.subcore_barrier()
  pltpu.sync_copy(numsh.at[pl.ds(sid * NSLC, NSLC)],
                  num_out.at[cid, pl.ds(sid * NSLC, NSLC)])


@functools.partial(
    pl.kernel,
    out_type=(_f32(KHOP, NC * NB, HF), _f32(NC * NB, HF), _f32(NC * NB, HF)),
    mesh=_mesh,
    compiler_params=pltpu.CompilerParams(needs_layout_passes=False),
    scratch_types=[
        pltpu.VMEM((4, CHC), jnp.int32),
        pltpu.VMEM((4, CHC), jnp.int32),
        pltpu.VMEM((NSLC,), jnp.float32),
        pltpu.VMEM((CHC, HF), jnp.float32),
        pltpu.VMEM((CHC, HF), jnp.float32),
        pltpu.VMEM((CHC, HF), jnp.float32),
        pltpu.VMEM((CHC, HF), jnp.float32),
        pltpu.SemaphoreType.DMA,
        pltpu.SemaphoreType.DMA,
        pltpu.SemaphoreType.DMA,
        pltpu.SemaphoreType.DMA,
        pltpu.SemaphoreType.DMA,
        pltpu.SemaphoreType.DMA,
        pltpu.SemaphoreType.DMA,
        pltpu.SemaphoreType.DMA,
        pltpu.SemaphoreType.DMA,
        pltpu.SemaphoreType.DMA,
        pltpu.SemaphoreType.DMA,
        pltpu.SemaphoreType.DMA,
        pltpu.VMEM_SHARED((NB, HF), jnp.float32),
    ],
)
def _dagnn(g0t, srcgC, dstC, normv, zeros_rows, f_all, gta, gtb, siv, div,
           normb, r0, r1, r2, r3, i0, i1, i2, i3, g0, g1, g2, g3, s0, s1,
           s2, s3, binsh):
  cid, sid, _ = _wid()
  pltpu.sync_copy(normv.at[pl.ds(sid * NSLC, NSLC)], normb)
  rbufs = (r0, r1, r2, r3)
  isems = (i0, i1, i2, i3)
  gsems = (g0, g1, g2, g3)
  ssems = (s0, s1, s2, s3)
  tabs = (g0t, gta, gtb)
  for r in range(KHOP):
    gtab = tabs[r]
    pltpu.sync_copy(zeros_rows, binsh.at[pl.ds(sid * NSLC, NSLC)])
    plsc.subcore_barrier()

    def quad(kk, carry, gtab=gtab):
      k = 4 * kk
      ia = []
      for t in range(4):
        ia.append((pltpu.async_copy(srcgC.at[cid, sid, k + t], siv.at[t],
                                    isems[t]),
                   pltpu.async_copy(dstC.at[sid, k + t], div.at[t],
                                    isems[t])))
      dg = []
      for t in range(4):
        ia[t][0].wait()
        ia[t][1].wait()
        dg.append(pltpu.async_copy(gtab.at[siv.at[t]], rbufs[t], gsems[t]))
      dsc = []
      for t in range(4):
        dg[t].wait()
        dsc.append(pltpu.async_copy(rbufs[t], binsh.at[div.at[t]], ssems[t],
                                    add=True))
      for t in range(4):
        dsc[t].wait()
      return carry

    lax.fori_loop(0, NCHC // 4, quad, 0)
    plsc.subcore_barrier()

    def slice_k(kk, carry):
      off = sid * NSLC + kk * CHC
      pltpu.sync_copy(binsh.at[pl.ds(off, CHC)], r0)

      def grp(g, carry2):
        nv = normb[pl.ds(kk * CHC + g * L, L)]

        def node(j, carry3):
          row = g * L + j
          sv = _vtake(nv, jnp.full((L,), j, jnp.int32))
          for m in range(HF // L):
            sl = pl.ds(m * L, L)
            v = r0[row, sl] * sv
            r1[row, sl] = v
            r0[row, sl] = v * sv
          return carry3

        return lax.fori_loop(0, L, node, carry2)

      lax.fori_loop(0, CHC // L, grp, 0)
      pltpu.sync_copy(r1, f_all.at[r, pl.ds(cid * NB + off, CHC)])
      if r == 0:
        pltpu.sync_copy(r0, gta.at[pl.ds(cid * NB + off, CHC)])
      elif r == 1:
        pltpu.sync_copy(r0, gtb.at[pl.ds(cid * NB + off, CHC)])
      return carry

    lax.fori_loop(0, NSLC // CHC, slice_k, 0)
    plsc.subcore_barrier()


@functools.partial(
    pl.kernel,
    out_type=_f32(BP),
    mesh=_mesh,
    compiler_params=pltpu.CompilerParams(needs_layout_passes=False),
    scratch_types=[
        pltpu.VMEM((CH,), jnp.int32),
        pltpu.VMEM((CH,), jnp.int32),
        pltpu.VMEM((CH, OUT), jnp.float32),
        pltpu.VMEM((CH, OUT), jnp.float32),
        pltpu.VMEM((OUT,), jnp.float32),
        pltpu.VMEM((OUT,), jnp.float32),
        pltpu.VMEM((L,), jnp.float32),
        pltpu.VMEM((CH,), jnp.float32),
        pltpu.SemaphoreType.DMA,
    ],
)
def _pair_scores(h, dis, mir, wpd, wpm, bp16, o_out, dib, mib, hrd, hrm,
                 wdb, wmb, bpb, obuf, sem):
  _, _, wid = _wid()
  pltpu.sync_copy(wpd, wdb)
  pltpu.sync_copy(wpm, wmb)
  pltpu.sync_copy(bp16, bpb)
  npairs = BP // NW

  def chunk(i, carry):
    base = wid * npairs + i * CH
    pltpu.sync_copy(dis.at[pl.ds(base, CH)], dib)
    pltpu.sync_copy(mir.at[pl.ds(base, CH)], mib)
    c1 = pltpu.async_copy(h.at[dib], hrd, sem)
    c2 = pltpu.async_copy(h.at[mib], hrm, sem)
    c1.wait()
    c2.wait()

    iota = _iota16()
    for g in range(CH // L):

      def pair(j, tv, _g=g):
        row = _g * L + j
        acc = jnp.zeros((L,), jnp.float32)
        for m in range(OUT // L):
          sl2 = pl.ds(m * L, L)
          acc = acc + hrd[row, sl2] * wdb[sl2]
          acc = acc + hrm[row, sl2] * wmb[sl2]
        return jnp.where(iota == j, jnp.sum(acc), tv)

      tv = lax.fori_loop(0, L, pair, jnp.zeros((L,), jnp.float32))
      t = tv + bpb[...]
      obuf[pl.ds(g * L, L)] = 1.0 / (1.0 + jnp.exp(-t))
    pltpu.sync_copy(obuf, o_out.at[pl.ds(base, CH)])
    return carry

  lax.fori_loop(0, npairs // CH, chunk, 0)


def _k1_body(d_ref, m_ref, wd_ref, wm_ref, zh_ref, z_ref):
  p = pl.program_id(0)
  is_d = (p % 10) < (ND // 1000)
  x = jnp.where(is_d, d_ref[...], m_ref[...])
  w = jnp.where(is_d, wd_ref[...], wm_ref[...])
  blk = jnp.dot(x, w, preferred_element_type=jnp.float32)
  zh_ref[...] = blk
  z_ref[...] = blk


def _node_transform(d_sim, m_sim, wd, wm):
  return pl.pallas_call(
      _k1_body,
      grid=(20,),
      in_specs=[
          pl.BlockSpec((1000, D), lambda g: (g % 10, 0)),
          pl.BlockSpec((1000, D), lambda g: (g % 10, 0)),
          pl.BlockSpec((D, HF), lambda g: (0, g // 10)),
          pl.BlockSpec((D, HF), lambda g: (0, g // 10)),
      ],
      out_specs=[
          pl.BlockSpec((1000, HF), lambda g: (g, 0)),
          pl.BlockSpec((1000, HF), lambda g: (g % 10, g // 10)),
      ],
      out_shape=(_f32(2 * N, HF), _f32(N, D)),
  )(d_sim, m_sim, wd, wm)


def _k2_body(mx_ref, dg_ref, c_ref, n_ref):
  c = jnp.max(mx_ref[...], axis=0)
  c_ref[...] = jnp.maximum(c, 0.0)
  deg = jnp.sum(dg_ref[...], axis=0)
  n_ref[...] = jnp.where(deg > 0, lax.rsqrt(jnp.maximum(deg, 1e-30)), 0.0)


def _combine_stats(maxpart, degpart):
  return pl.pallas_call(
      _k2_body,
      out_shape=(_f32(NB), _f32(NB)),
  )(maxpart, degpart)


def _k3_body(num_ref, den_ref, nrm_ref, ft_ref, g0_ref):
  i = pl.program_id(0)
  den = jnp.sum(den_ref[:, pl.ds(i % 8 * 1280, 1280)], axis=0)
  nrm = nrm_ref[pl.ds(i % 8 * 1280, 1280)]
  den = jnp.where(den > 0, den, 1.0)
  feats = _elu(num_ref[...] / den[:, None])
  ft_ref[...] = feats
  g0_ref[...] = feats * nrm[:, None]


def _feats_g0(num, den, normv):
  return pl.pallas_call(
      _k3_body,
      grid=(16,),
      in_specs=[
          pl.BlockSpec((1280, HF), lambda i: (i, 0)),
          pl.BlockSpec((NW, NB), lambda i: (0, 0)),
          pl.BlockSpec((NB,), lambda i: (0,)),
      ],
      out_specs=[
          pl.BlockSpec((1280, HF), lambda i: (i, 0)),
          pl.BlockSpec((1280, HF), lambda i: (i, 0)),
      ],
      out_shape=(_f32(NC * NB, HF), _f32(NC * NB, HF)),
  )(num.reshape(NC * NB, HF), den, normv)


def _k4_body(ft_ref, f1_ref, f2_ref, f3_ref, d_ref, m_ref, s_ref, wd_ref,
             bd_ref, wm_ref, bm_ref, h_ref):
  p = pl.program_id(0)
  hout = jnp.zeros((1000, F), jnp.float32)
  for ref in (ft_ref, f1_ref, f2_ref, f3_ref):
    hk = jnp.concatenate([ref[0], ref[1]], axis=1)
    sk = _sigmoid(jnp.dot(hk, s_ref[...], precision=lax.Precision.HIGHEST,
                          preferred_element_type=jnp.float32))
    hout = hout + sk[:, None] * hk
  is_d = p < (ND // 1000)
  sim = jnp.where(is_d, d_ref[...], m_ref[...])
  w = jnp.where(is_d, wd_ref[...], wm_ref[...])
  b = jnp.where(is_d, bd_ref[...], bm_ref[...])
  x = jnp.concatenate([hout, sim], axis=1)
  h_ref[...] = _elu(
      jnp.dot(x, w, precision=lax.Precision.HIGHEST,
              preferred_element_type=jnp.float32) + b[None, :])


def _final_mlp(ft, f1, f2, f3, d_sim, m_sim, s, wd_fc, bd_fc, wm_fc, bm_fc):
  fspec = pl.BlockSpec((NC, 1000, HF), lambda i: (0, i, 0))
  return pl.pallas_call(
      _k4_body,
      grid=(10,),
      in_specs=[
          fspec, fspec, fspec, fspec,
          pl.BlockSpec((1000, D), lambda i: (i, 0)),
          pl.BlockSpec((1000, D), lambda i: (i, 0)),
          pl.BlockSpec((F,), lambda i: (0,)),
          pl.BlockSpec((F + D, OUT), lambda i: (0, 0)),
          pl.BlockSpec((OUT,), lambda i: (0,)),
          pl.BlockSpec((F + D, OUT), lambda i: (0, 0)),
          pl.BlockSpec((OUT,), lambda i: (0,)),
      ],
      out_specs=pl.BlockSpec((1000, OUT), lambda i: (i, 0)),
      out_shape=_f32(N, OUT),
  )(ft, f1, f2, f3, d_sim, m_sim, s, wd_fc, bd_fc, wm_fc, bm_fc)


def kernel(d_sim, m_sim, W_d1, W_m1, W_d2, W_m2, s, Wd_fc, bd_fc, Wm_fc,
           bm_fc, Wp, bp, edge_index, diseases, mirnas):
  src = edge_index[0]
  dst = edge_index[1]
  pad = EPAD - E
  src_pad = jnp.concatenate([src, jnp.zeros((pad,), jnp.int32)])
  dst_pad = jnp.concatenate([dst, jnp.full((pad,), N, jnp.int32)])
  dstg = jnp.minimum(dst_pad, N - 1)
  srcA3 = src_pad.reshape(NW, NCHA, CHA)
  dgA3 = dstg.reshape(NW, NCHA, CHA)
  dstA3 = dst_pad.reshape(NW, NCHA, CHA)
  srczB = jnp.stack([src_pad, src_pad + N]).reshape(NC, NS, NCHB, CHB)
  dstB3 = dst_pad.reshape(NS, NCHB, CHB)
  dstE3 = dst_pad.reshape(NW, NCHA, CHA)
  srcgC = jnp.stack([src_pad, src_pad + NB]).reshape(NC, NS, NCHC, CHC)
  dstC3 = dst_pad.reshape(NS, NCHC, CHC)
  zeros_nb = jnp.zeros((NB,), jnp.float32)
  zeros_rows = jnp.zeros((NSLC, HF), jnp.float32)
  wpd = Wp[:OUT, 0]
  wpm = Wp[OUT:, 0]
  bp16 = jnp.full((L,), 0.0, jnp.float32) + bp[0]

  zh, z = _node_transform(d_sim, m_sim, W_d2, W_m2)
  e_pad, maxpart, degpart = _edge_scores(z, srcA3, dgA3, dstA3, zeros_nb)
  cvec, normv = _combine_stats(maxpart, degpart)
  exfull, denpart = _edge_weights(e_pad, dstE3, cvec, zeros_nb)
  num = _attn_aggregate(zh, srczB, dstB3,
                        exfull.reshape(NS, NCHB, CHB), zeros_rows)
  feats, g0 = _feats_g0(num, denpart, normv)
  f_all, _, _ = _dagnn(g0, srcgC, dstC3, normv, zeros_rows)
  ftr = feats.reshape(NC, NB, HF)
  f1 = f_all[0].reshape(NC, NB, HF)
  f2 = f_all[1].reshape(NC, NB, HF)
  f3 = f_all[2].reshape(NC, NB, HF)
  h = _final_mlp(ftr, f1, f2, f3, d_sim, m_sim, s[:, 0], Wd_fc, bd_fc,
                 Wm_fc, bm_fc)
  o = _pair_scores(h, diseases, mirnas, wpd, wpm, bp16)
  return o.reshape(BP, 1)


# speedup vs baseline: 3.2608x; 1.0306x over previous
import functools

import jax
import jax.numpy as jnp
from jax import lax
from jax.experimental import pallas as pl
from jax.experimental.pallas import tpu as pltpu
from jax.experimental.pallas import tpu_sc as plsc

N = 10000
ND = 4000
E = 160000
D = 256
F = 256
OUT = 128
KHOP = 3
BP = 16384
SLOPE = 0.2

NC = 2
NS = 16
L = 16
NW = NC * NS

EPAD = 163840
EWA = EPAD // NW
EWS = EPAD // NS
CH = 64
CHA = 64
NCHA = EWA // CHA
CHB = 64
NCHB = EWS // CHB
CHC = 64
NCHC = EWS // CHC
NB = 10240
NSLC = NB // NS
HF = 128

_mesh = plsc.VectorSubcoreMesh(
    core_axis_name="c", subcore_axis_name="s", num_cores=NC, num_subcores=NS)


def _elu(x):
  return jnp.where(x > 0, x, jnp.exp(jnp.minimum(x, 0.0)) - 1.0)


def _sigmoid(x):
  return 1.0 / (1.0 + jnp.exp(-x))


def _f32(*shape):
  return jax.ShapeDtypeStruct(shape, jnp.float32)


def _wid():
  c = lax.axis_index("c")
  s = lax.axis_index("s")
  return c, s, c * NS + s


def _iota16():
  return lax.broadcasted_iota(jnp.int32, (L,), 0)


def _vtake(x, idx):
  return x.at[idx].get(mode="promise_in_bounds")


def _runs(ks, vs, combine):
  iota = _iota16()
  for sh in (1, 2, 4, 8):
    pidx = jnp.maximum(iota - sh, 0)
    kp = _vtake(ks, pidx)
    vp = _vtake(vs, pidx)
    vs = jnp.where((kp == ks) & (iota >= sh), combine(vs, vp), vs)
  knext = _vtake(ks, jnp.minimum(iota + 1, L - 1))
  is_last = (ks != knext) | (iota == L - 1)
  return vs, is_last


@functools.partial(
    pl.kernel,
    out_type=(_f32(EPAD), _f32(NW, NB), _f32(NW, NB)),
    mesh=_mesh,
    compiler_params=pltpu.CompilerParams(needs_layout_passes=False),
    scratch_types=[
        pltpu.VMEM((NCHA, CHA), jnp.int32),
        pltpu.VMEM((NCHA, CHA), jnp.int32),
        pltpu.VMEM((NCHA, CHA), jnp.int32),
        pltpu.VMEM((EWA,), jnp.float32),
        pltpu.VMEM((NB,), jnp.float32),
        pltpu.VMEM((NB,), jnp.float32),
        pltpu.VMEM((CHA, D), jnp.float32),
        pltpu.VMEM((CHA, D), jnp.float32),
        pltpu.VMEM((CHA, D), jnp.float32),
        pltpu.VMEM((CHA, D), jnp.float32),
        pltpu.SemaphoreType.DMA,
        pltpu.SemaphoreType.DMA,
    ],
)
def _edge_scores(z, srcA, dgA, dstA, zeros_nb, e_out, maxpart, degpart,
                 sA, gA, kA, ebuf, mbins, dbins, sr0, dr0, sr1, dr1, g0, g1):
  cid, sid, wid = _wid()
  pltpu.sync_copy(srcA.at[wid], sA)
  pltpu.sync_copy(dgA.at[wid], gA)
  pltpu.sync_copy(dstA.at[wid], kA)
  pltpu.sync_copy(zeros_nb, mbins)
  pltpu.sync_copy(zeros_nb, dbins)
  iota = _iota16()
  ones = jnp.ones((L,), jnp.float32)

  def process(k, sr, dr):
    def grp(g, carry2):
      def edge(j, ev):
        row = g * L + j
        acc = jnp.zeros((L,), jnp.float32)
        for m in range(D // L):
          sl2 = pl.ds(m * L, L)
          acc = acc + sr[row, sl2] * dr[row, sl2]
        dot = jnp.sum(acc)
        e = jnp.where(dot > 0, dot, SLOPE * dot)
        return jnp.where(iota == j, e, ev)

      ev = lax.fori_loop(0, L, edge, jnp.zeros((L,), jnp.float32))
      ebuf[pl.ds(k * CHA + g * L, L)] = ev
      ks, vs = plsc.sort_key_val(kA[k, pl.ds(g * L, L)], ev)
      vmax, is_last = _runs(ks, vs, jnp.maximum)
      cur = plsc.load_gather(mbins, [ks])
      plsc.store_scatter(mbins, [ks], jnp.maximum(cur, vmax), mask=is_last)
      cnt, _ = _runs(ks, ones, lambda a, b: a + b)
      dcur = plsc.load_gather(dbins, [ks])
      plsc.store_scatter(dbins, [ks], dcur + cnt, mask=is_last)
      return carry2

    lax.fori_loop(0, CHA // L, grp, 0)

  def pair(kk, carry):
    k0 = 2 * kk
    k1 = k0 + 1
    ds0 = pltpu.async_copy(z.at[sA.at[k0]], sr0, g0)
    dd0 = pltpu.async_copy(z.at[gA.at[k0]], dr0, g0)
    ds1 = pltpu.async_copy(z.at[sA.at[k1]], sr1, g1)
    dd1 = pltpu.async_copy(z.at[gA.at[k1]], dr1, g1)
    ds0.wait()
    dd0.wait()
    process(k0, sr0, dr0)
    ds1.wait()
    dd1.wait()
    process(k1, sr1, dr1)
    return carry

  lax.fori_loop(0, NCHA // 2, pair, 0)
  pltpu.sync_copy(ebuf, e_out.at[pl.ds(wid * EWA, EWA)])
  pltpu.sync_copy(mbins, maxpart.at[wid])
  pltpu.sync_copy(dbins, degpart.at[wid])


@functools.partial(
    pl.kernel,
    out_type=(_f32(EPAD), _f32(NW, NB)),
    mesh=_mesh,
    compiler_params=pltpu.CompilerParams(needs_layout_passes=False),
    scratch_types=[
        pltpu.VMEM((NCHA, CHA), jnp.int32),
        pltpu.VMEM((EWA,), jnp.float32),
        pltpu.VMEM((EWA,), jnp.float32),
        pltpu.VMEM((NB,), jnp.float32),
        pltpu.VMEM((NB,), jnp.float32),
    ],
)
def _edge_weights(e_in, dstA, cvec, zeros_nb, ex_out, denpart, kA, e1, exb,
                  cbins, dbins):
  cid, sid, wid = _wid()
  pltpu.sync_copy(dstA.at[wid], kA)
  pltpu.sync_copy(e_in.at[pl.ds(wid * EWA, EWA)], e1)
  pltpu.sync_copy(cvec, cbins)
  pltpu.sync_copy(zeros_nb, dbins)

  def chunk(k, carry):
    def grp(g, carry2):
      dk = kA[k, pl.ds(g * L, L)]
      evv = e1[pl.ds(k * CHA + g * L, L)]
      cg = plsc.load_gather(cbins, [dk])
      exv = jnp.exp(evv - cg)
      exb[pl.ds(k * CHA + g * L, L)] = exv
      ks, xs = plsc.sort_key_val(dk, exv)
      ssum, is_last = _runs(ks, xs, lambda a, b: a + b)
      cur = plsc.load_gather(dbins, [ks])
      plsc.store_scatter(dbins, [ks], cur + ssum, mask=is_last)
      return carry2

    return lax.fori_loop(0, CHA // L, grp, carry)

  lax.fori_loop(0, NCHA, chunk, 0)
  pltpu.sync_copy(exb, ex_out.at[pl.ds(wid * EWA, EWA)])
  pltpu.sync_copy(dbins, denpart.at[wid])


@functools.partial(
    pl.kernel,
    out_type=_f32(NC, NB, HF),
    mesh=_mesh,
    compiler_params=pltpu.CompilerParams(needs_layout_passes=False),
    scratch_types=[
        pltpu.VMEM((4, CHB), jnp.int32),
        pltpu.VMEM((4, CHB), jnp.int32),
        pltpu.VMEM((4, CHB), jnp.float32),
        pltpu.VMEM((CHB, HF), jnp.float32),
        pltpu.VMEM((CHB, HF), jnp.float32),
        pltpu.VMEM((CHB, HF), jnp.float32),
        pltpu.VMEM((CHB, HF), jnp.float32),
        pltpu.SemaphoreType.DMA,
        pltpu.SemaphoreType.DMA,
        pltpu.SemaphoreType.DMA,
        pltpu.SemaphoreType.DMA,
        pltpu.SemaphoreType.DMA,
        pltpu.SemaphoreType.DMA,
        pltpu.SemaphoreType.DMA,
        pltpu.SemaphoreType.DMA,
        pltpu.SemaphoreType.DMA,
        pltpu.SemaphoreType.DMA,
        pltpu.SemaphoreType.DMA,
        pltpu.SemaphoreType.DMA,
        pltpu.VMEM_SHARED((NB, HF), jnp.float32),
    ],
)
def _attn_aggregate(zh, srczB, dstB, exB, zeros_rows, num_out, siv, div,
                    exv, r0, r1, r2, r3, i0, i1, i2, i3, g0, g1, g2, g3,
                    s0, s1, s2, s3, numsh):
  cid, sid, wid = _wid()
  pltpu.sync_copy(zeros_rows, numsh.at[pl.ds(sid * NSLC, NSLC)])
  plsc.subcore_barrier()
  rbufs = (r0, r1, r2, r3)
  isems = (i0, i1, i2, i3)
  gsems = (g0, g1, g2, g3)
  ssems = (s0, s1, s2, s3)

  def scale(r, t):
    def grp(g, carry2):
      ex16 = exv[t, pl.ds(g * L, L)]

      def edge(j, carry3):
        row = g * L + j
        sv = _vtake(ex16, jnp.full((L,), j, jnp.int32))
        for m in range(HF // L):
          sl2 = pl.ds(m * L, L)
          r[row, sl2] = r[row, sl2] * sv
        return carry3

      return lax.fori_loop(0, L, edge, carry2)

    lax.fori_loop(0, CHB // L, grp, 0)

  def quad(kk, carry):
    k = 4 * kk
    ia = []
    for t in range(4):
      ia.append((pltpu.async_copy(srczB.at[cid, sid, k + t], siv.at[t],
                                  isems[t]),
                 pltpu.async_copy(dstB.at[sid, k + t], div.at[t], isems[t]),
                 pltpu.async_copy(exB.at[sid, k + t], exv.at[t], isems[t])))
    dg = []
    for t in range(4):
      ia[t][0].wait()
      ia[t][1].wait()
      ia[t][2].wait()
      dg.append(pltpu.async_copy(zh.at[siv.at[t]], rbufs[t], gsems[t]))
    dsc = []
    for t in range(4):
      dg[t].wait()
      scale(rbufs[t], t)
      dsc.append(pltpu.async_copy(rbufs[t], numsh.at[div.at[t]], ssems[t],
                                  add=True))
    for t in range(4):
      dsc[t].wait()
    return carry

  lax.fori_loop(0, NCHB // 4, quad, 0)
  plsc.subcore_barrier()
  pltpu.sync_copy(numsh.at[pl.ds(sid * NSLC, NSLC)],
                  num_out.at[cid, pl.ds(sid * NSLC, NSLC)])


@functools.partial(
    pl.kernel,
    out_type=(_f32(KHOP, NC * NB, HF), _f32(NC * NB, HF), _f32(NC * NB, HF)),
    mesh=_mesh,
    compiler_params=pltpu.CompilerParams(needs_layout_passes=False),
    scratch_types=[
        pltpu.VMEM((4, CHC), jnp.int32),
        pltpu.VMEM((4, CHC), jnp.int32),
        pltpu.VMEM((NSLC,), jnp.float32),
        pltpu.VMEM((CHC, HF), jnp.float32),
        pltpu.VMEM((CHC, HF), jnp.float32),
        pltpu.VMEM((CHC, HF), jnp.float32),
        pltpu.VMEM((CHC, HF), jnp.float32),
        pltpu.SemaphoreType.DMA,
        pltpu.SemaphoreType.DMA,
        pltpu.SemaphoreType.DMA,
        pltpu.SemaphoreType.DMA,
        pltpu.SemaphoreType.DMA,
        pltpu.SemaphoreType.DMA,
        pltpu.SemaphoreType.DMA,
        pltpu.SemaphoreType.DMA,
        pltpu.SemaphoreType.DMA,
        pltpu.SemaphoreType.DMA,
        pltpu.SemaphoreType.DMA,
        pltpu.SemaphoreType.DMA,
        pltpu.VMEM_SHARED((NB, HF), jnp.float32),
    ],
)
def _dagnn(g0t, srcgC, dstC, normv, zeros_rows, f_all, gta, gtb, siv, div,
           normb, r0, r1, r2, r3, i0, i1, i2, i3, g0, g1, g2, g3, s0, s1,
           s2, s3, binsh):
  cid, sid, _ = _wid()
  pltpu.sync_copy(normv.at[pl.ds(sid * NSLC, NSLC)], normb)
  rbufs = (r0, r1, r2, r3)
  isems = (i0, i1, i2, i3)
  gsems = (g0, g1, g2, g3)
  ssems = (s0, s1, s2, s3)
  tabs = (g0t, gta, gtb)
  for r in range(KHOP):
    gtab = tabs[r]
    pltpu.sync_copy(zeros_rows, binsh.at[pl.ds(sid * NSLC, NSLC)])
    plsc.subcore_barrier()

    def quad(kk, carry, gtab=gtab):
      k = 4 * kk
      ia = []
      for t in range(4):
        ia.append((pltpu.async_copy(srcgC.at[cid, sid, k + t], siv.at[t],
                                    isems[t]),
                   pltpu.async_copy(dstC.at[sid, k + t], div.at[t],
                                    isems[t])))
      dg = []
      for t in range(4):
        ia[t][0].wait()
        ia[t][1].wait()
        dg.append(pltpu.async_copy(gtab.at[siv.at[t]], rbufs[t], gsems[t]))
      dsc = []
      for t in range(4):
        dg[t].wait()
        dsc.append(pltpu.async_copy(rbufs[t], binsh.at[div.at[t]], ssems[t],
                                    add=True))
      for t in range(4):
        dsc[t].wait()
      return carry

    lax.fori_loop(0, NCHC // 4, quad, 0)
    plsc.subcore_barrier()

    def slice_k(kk, carry):
      off = sid * NSLC + kk * CHC
      pltpu.sync_copy(binsh.at[pl.ds(off, CHC)], r0)

      def grp(g, carry2):
        nv = normb[pl.ds(kk * CHC + g * L, L)]

        def node(j, carry3):
          row = g * L + j
          sv = _vtake(nv, jnp.full((L,), j, jnp.int32))
          for m in range(HF // L):
            sl = pl.ds(m * L, L)
            v = r0[row, sl] * sv
            r1[row, sl] = v
            r0[row, sl] = v * sv
          return carry3

        return lax.fori_loop(0, L, node, carry2)

      lax.fori_loop(0, CHC // L, grp, 0)
      pltpu.sync_copy(r1, f_all.at[r, pl.ds(cid * NB + off, CHC)])
      if r == 0:
        pltpu.sync_copy(r0, gta.at[pl.ds(cid * NB + off, CHC)])
      elif r == 1:
        pltpu.sync_copy(r0, gtb.at[pl.ds(cid * NB + off, CHC)])
      return carry

    lax.fori_loop(0, NSLC // CHC, slice_k, 0)
    plsc.subcore_barrier()


@functools.partial(
    pl.kernel,
    out_type=_f32(BP),
    mesh=_mesh,
    compiler_params=pltpu.CompilerParams(needs_layout_passes=False),
    scratch_types=[
        pltpu.VMEM((CH,), jnp.int32),
        pltpu.VMEM((CH,), jnp.int32),
        pltpu.VMEM((CH, OUT), jnp.float32),
        pltpu.VMEM((CH, OUT), jnp.float32),
        pltpu.VMEM((OUT,), jnp.float32),
        pltpu.VMEM((OUT,), jnp.float32),
        pltpu.VMEM((L,), jnp.float32),
        pltpu.VMEM((CH,), jnp.float32),
        pltpu.SemaphoreType.DMA,
    ],
)
def _pair_scores(h, dis, mir, wpd, wpm, bp16, o_out, dib, mib, hrd, hrm,
                 wdb, wmb, bpb, obuf, sem):
  _, _, wid = _wid()
  pltpu.sync_copy(wpd, wdb)
  pltpu.sync_copy(wpm, wmb)
  pltpu.sync_copy(bp16, bpb)
  npairs = BP // NW

  def chunk(i, carry):
    base = wid * npairs + i * CH
    pltpu.sync_copy(dis.at[pl.ds(base, CH)], dib)
    pltpu.sync_copy(mir.at[pl.ds(base, CH)], mib)
    c1 = pltpu.async_copy(h.at[dib], hrd, sem)
    c2 = pltpu.async_copy(h.at[mib], hrm, sem)
    c1.wait()
    c2.wait()

    iota = _iota16()
    for g in range(CH // L):

      def pair(j, tv, _g=g):
        row = _g * L + j
        acc = jnp.zeros((L,), jnp.float32)
        for m in range(OUT // L):
          sl2 = pl.ds(m * L, L)
          acc = acc + hrd[row, sl2] * wdb[sl2]
          acc = acc + hrm[row, sl2] * wmb[sl2]
        return jnp.where(iota == j, jnp.sum(acc), tv)

      tv = lax.fori_loop(0, L, pair, jnp.zeros((L,), jnp.float32))
      t = tv + bpb[...]
      obuf[pl.ds(g * L, L)] = 1.0 / (1.0 + jnp.exp(-t))
    pltpu.sync_copy(obuf, o_out.at[pl.ds(base, CH)])
    return carry

  lax.fori_loop(0, npairs // CH, chunk, 0)


def _k1_body(d_ref, m_ref, wd_ref, wm_ref, zh_ref, z_ref):
  p = pl.program_id(0)
  is_d = (p % 10) < (ND // 1000)
  x = jnp.where(is_d, d_ref[...], m_ref[...])
  w = jnp.where(is_d, wd_ref[...], wm_ref[...])
  blk = jnp.dot(x, w, preferred_element_type=jnp.float32)
  zh_ref[...] = blk
  z_ref[...] = blk


def _node_transform(d_sim, m_sim, wd, wm):
  return pl.pallas_call(
      _k1_body,
      grid=(20,),
      in_specs=[
          pl.BlockSpec((1000, D), lambda g: (g % 10, 0)),
          pl.BlockSpec((1000, D), lambda g: (g % 10, 0)),
          pl.BlockSpec((D, HF), lambda g: (0, g // 10)),
          pl.BlockSpec((D, HF), lambda g: (0, g // 10)),
      ],
      out_specs=[
          pl.BlockSpec((1000, HF), lambda g: (g, 0)),
          pl.BlockSpec((1000, HF), lambda g: (g % 10, g // 10)),
      ],
      out_shape=(_f32(2 * N, HF), _f32(N, D)),
  )(d_sim, m_sim, wd, wm)


def _k2_body(mx_ref, dg_ref, c_ref, n_ref):
  c = jnp.max(mx_ref[...], axis=0)
  c_ref[...] = jnp.maximum(c, 0.0)
  deg = jnp.sum(dg_ref[...], axis=0)
  n_ref[...] = jnp.where(deg > 0, lax.rsqrt(jnp.maximum(deg, 1e-30)), 0.0)


def _combine_stats(maxpart, degpart):
  return pl.pallas_call(
      _k2_body,
      out_shape=(_f32(NB), _f32(NB)),
  )(maxpart, degpart)


def _k3_body(num_ref, den_ref, nrm_ref, ft_ref, g0_ref):
  i = pl.program_id(0)
  den = jnp.sum(den_ref[:, pl.ds(i % 8 * 1280, 1280)], axis=0)
  nrm = nrm_ref[pl.ds(i % 8 * 1280, 1280)]
  den = jnp.where(den > 0, den, 1.0)
  feats = _elu(num_ref[...] / den[:, None])
  ft_ref[...] = feats
  g0_ref[...] = feats * nrm[:, None]


def _feats_g0(num, den, normv):
  return pl.pallas_call(
      _k3_body,
      grid=(16,),
      in_specs=[
          pl.BlockSpec((1280, HF), lambda i: (i, 0)),
          pl.BlockSpec((NW, NB), lambda i: (0, 0)),
          pl.BlockSpec((NB,), lambda i: (0,)),
      ],
      out_specs=[
          pl.BlockSpec((1280, HF), lambda i: (i, 0)),
          pl.BlockSpec((1280, HF), lambda i: (i, 0)),
      ],
      out_shape=(_f32(NC * NB, HF), _f32(NC * NB, HF)),
  )(num.reshape(NC * NB, HF), den, normv)


def _k4_body(ft_ref, f1_ref, f2_ref, f3_ref, d_ref, m_ref, s_ref, wd_ref,
             bd_ref, wm_ref, bm_ref, h_ref):
  p = pl.program_id(0)
  hout = jnp.zeros((1000, F), jnp.float32)
  for ref in (ft_ref, f1_ref, f2_ref, f3_ref):
    hk = jnp.concatenate([ref[0], ref[1]], axis=1)
    sk = _sigmoid(jnp.dot(hk, s_ref[...], precision=lax.Precision.HIGHEST,
                          preferred_element_type=jnp.float32))
    hout = hout + sk[:, None] * hk
  is_d = p < (ND // 1000)
  sim = jnp.where(is_d, d_ref[...], m_ref[...])
  w = jnp.where(is_d, wd_ref[...], wm_ref[...])
  b = jnp.where(is_d, bd_ref[...], bm_ref[...])
  x = jnp.concatenate([hout, sim], axis=1)
  h_ref[...] = _elu(
      jnp.dot(x, w, precision=lax.Precision.HIGHEST,
              preferred_element_type=jnp.float32) + b[None, :])


def _final_mlp(ft, f1, f2, f3, d_sim, m_sim, s, wd_fc, bd_fc, wm_fc, bm_fc):
  fspec = pl.BlockSpec((NC, 1000, HF), lambda i: (0, i, 0))
  return pl.pallas_call(
      _k4_body,
      grid=(10,),
      in_specs=[
          fspec, fspec, fspec, fspec,
          pl.BlockSpec((1000, D), lambda i: (i, 0)),
          pl.BlockSpec((1000, D), lambda i: (i, 0)),
          pl.BlockSpec((F,), lambda i: (0,)),
          pl.BlockSpec((F + D, OUT), lambda i: (0, 0)),
          pl.BlockSpec((OUT,), lambda i: (0,)),
          pl.BlockSpec((F + D, OUT), lambda i: (0, 0)),
          pl.BlockSpec((OUT,), lambda i: (0,)),
      ],
      out_specs=pl.BlockSpec((1000, OUT), lambda i: (i, 0)),
      out_shape=_f32(N, OUT),
  )(ft, f1, f2, f3, d_sim, m_sim, s, wd_fc, bd_fc, wm_fc, bm_fc)


def kernel(d_sim, m_sim, W_d1, W_m1, W_d2, W_m2, s, Wd_fc, bd_fc, Wm_fc,
           bm_fc, Wp, bp, edge_index, diseases, mirnas):
  src = edge_index[0]
  dst = edge_index[1]
  pad = EPAD - E
  src_pad = jnp.concatenate([src, jnp.zeros((pad,), jnp.int32)])
  dst_pad = jnp.concatenate([dst, jnp.full((pad,), N, jnp.int32)])
  dstg = jnp.minimum(dst_pad, N - 1)
  srcA3 = src_pad.reshape(NW, NCHA, CHA)
  dgA3 = dstg.reshape(NW, NCHA, CHA)
  dstA3 = dst_pad.reshape(NW, NCHA, CHA)
  srczB = jnp.stack([src_pad, src_pad + N]).reshape(NC, NS, NCHB, CHB)
  dstB3 = dst_pad.reshape(NS, NCHB, CHB)
  dstE3 = dst_pad.reshape(NW, NCHA, CHA)
  srcgC = jnp.stack([src_pad, src_pad + NB]).reshape(NC, NS, NCHC, CHC)
  dstC3 = dst_pad.reshape(NS, NCHC, CHC)
  zeros_nb = jnp.zeros((NB,), jnp.float32)
  zeros_rows = jnp.zeros((NSLC, HF), jnp.float32)
  wpd = Wp[:OUT, 0]
  wpm = Wp[OUT:, 0]
  bp16 = jnp.full((L,), 0.0, jnp.float32) + bp[0]

  zh, z = _node_transform(d_sim, m_sim, W_d2, W_m2)
  e_pad, maxpart, degpart = _edge_scores(z, srcA3, dgA3, dstA3, zeros_nb)
  cvec, normv = _combine_stats(maxpart, degpart)
  exfull, denpart = _edge_weights(e_pad, dstE3, cvec, zeros_nb)
  num = _attn_aggregate(zh, srczB, dstB3,
                        exfull.reshape(NS, NCHB, CHB), zeros_rows)
  feats, g0 = _feats_g0(num, denpart, normv)
  f_all, _, _ = _dagnn(g0, srcgC, dstC3, normv, zeros_rows)
  ftr = feats.reshape(NC, NB, HF)
  f1 = f_all[0].reshape(NC, NB, HF)
  f2 = f_all[1].reshape(NC, NB, HF)
  f3 = f_all[2].reshape(NC, NB, HF)
  h = _final_mlp(ftr, f1, f2, f3, d_sim, m_sim, s[:, 0], Wd_fc, bd_fc,
                 Wm_fc, bm_fc)
  o = _pair_scores(h, diseases, mirnas, wpd, wpm, bp16)
  return o.reshape(BP, 1)
